# Initial kernel scaffold; baseline (speedup 1.0000x reference)
import jax
import jax.numpy as jnp
from jax.experimental import pallas as pl


def kernel(h, attention_weights, W1, b1, g1, beta1, W2, b2, g2, beta2, W3, b3, eta, pos_table):
    raise NotImplementedError("write your pallas kernel here")


# speedup vs baseline: 4.3390x; 4.3390x over previous
import functools

import jax
import jax.numpy as jnp
from jax import lax
from jax.experimental import pallas as pl
from jax.experimental.pallas import tpu as pltpu

LAM = 0.01
NEG_INF = float("-inf")


def _topk_body(attn_ref, vals_ref, idx_ref, *, blk, L):
    r = pl.program_id(0)
    x = attn_ref[...]
    rows = r * blk + lax.broadcasted_iota(jnp.int32, (blk, L), 0)
    cols = lax.broadcasted_iota(jnp.int32, (blk, L), 1)
    cur = jnp.where(cols < rows, x, NEG_INF)
    vs, ids = [], []
    for _ in range(4):
        m = jnp.max(cur, axis=1, keepdims=True)
        cand = jnp.where(cur == m, cols, L)
        a = jnp.min(cand, axis=1, keepdims=True)
        a = jnp.where(a == L, 0, a)
        vs.append(m)
        ids.append(a)
        cur = jnp.where(cols == a, NEG_INF, cur)
    chain_v = jnp.sum(jnp.where(cols == rows - 1, x, 0.0), axis=1, keepdims=True)
    chain_t = jnp.maximum(rows[:, 0:1] - 1, 0)
    zf = jnp.zeros((blk, 1), jnp.float32)
    zi = jnp.zeros((blk, 1), jnp.int32)
    vals_ref[...] = jnp.concatenate(vs + [chain_v, zf, zf, zf], axis=1)
    idx_ref[...] = jnp.concatenate(ids + [chain_t, zi, zi, zi], axis=1)


def _topk(attn):
    L = attn.shape[0]
    blk = min(256, L)
    return pl.pallas_call(
        functools.partial(_topk_body, blk=blk, L=L),
        grid=(L // blk,),
        in_specs=[pl.BlockSpec((blk, L), lambda r: (r, 0))],
        out_specs=[pl.BlockSpec((blk, 8), lambda r: (r, 0)),
                   pl.BlockSpec((blk, 8), lambda r: (r, 0))],
        out_shape=[jax.ShapeDtypeStruct((L, 8), jnp.float32),
                   jax.ShapeDtypeStruct((L, 8), jnp.int32)],
    )(attn)


def _wix_body(vals_ref, idx_ref, w_ref, iij_ref, iji_ref, *, L, maxrel):
    vals = vals_ref[...]
    t8 = idx_ref[...]
    lane = lax.broadcasted_iota(jnp.int32, (L, 8), 1)
    rowv = lax.broadcasted_iota(jnp.int32, (L, 8), 0)
    m = jnp.max(vals, axis=0, keepdims=True)
    e = jnp.exp(vals - m)
    s = jnp.sum(e, axis=0, keepdims=True)
    wsm = e / s
    w_ref[...] = jnp.where(lane < 4, wsm, jnp.where(lane == 4, vals, 0.0))
    rel = t8 - rowv
    iij_ref[...] = jnp.clip(rel, -maxrel, maxrel) + maxrel
    iji_ref[...] = jnp.clip(-rel, -maxrel, maxrel) + maxrel


def _wix(vals8, idx8, maxrel):
    L = vals8.shape[0]
    return pl.pallas_call(
        functools.partial(_wix_body, L=L, maxrel=maxrel),
        in_specs=[pl.BlockSpec((L, 8), lambda: (0, 0))] * 2,
        out_specs=[pl.BlockSpec((L, 8), lambda: (0, 0))] * 3,
        out_shape=[jax.ShapeDtypeStruct((L, 8), jnp.float32),
                   jax.ShapeDtypeStruct((L, 8), jnp.int32),
                   jax.ShapeDtypeStruct((L, 8), jnp.int32)],
    )(vals8, idx8)


def _mm_body(x_ref, w_ref, b_ref, o_ref):
    x = x_ref[...].astype(jnp.bfloat16)
    o_ref[...] = jnp.dot(x, w_ref[...], preferred_element_type=jnp.float32) + b_ref[...]


def _mm(x, wbf, bias):
    M, K = x.shape
    N = wbf.shape[1]
    bm = min(256, M)
    grid = (pl.cdiv(M, bm),)
    return pl.pallas_call(
        _mm_body,
        grid=grid,
        in_specs=[pl.BlockSpec((bm, K), lambda r: (r, 0)),
                  pl.BlockSpec((K, N), lambda r: (0, 0)),
                  pl.BlockSpec((1, N), lambda r: (0, 0))],
        out_specs=pl.BlockSpec((bm, N), lambda r: (r, 0)),
        out_shape=jax.ShapeDtypeStruct((M, N), jnp.float32),
    )(x, wbf, bias)


def _ln(x, g, b):
    mu = jnp.mean(x, axis=-1, keepdims=True)
    v = jnp.mean((x - mu) ** 2, axis=-1, keepdims=True)
    return (x - mu) * lax.rsqrt(v + 1e-5) * g + b


def _mlp_body(base_ref, gt_ref, gp_ref, w2_ref, b2_ref, w3_ref, b3_ref,
              g1_ref, be1_ref, g2_ref, be2_ref, v_ref):
    x1 = base_ref[...] + gt_ref[0] + gp_ref[0]
    u = jax.nn.relu(_ln(x1, g1_ref[...], be1_ref[...])).astype(jnp.bfloat16)
    h2 = jnp.dot(u, w2_ref[...], preferred_element_type=jnp.float32) + b2_ref[...]
    u2 = jax.nn.relu(_ln(h2, g2_ref[...], be2_ref[...])).astype(jnp.bfloat16)
    v = jnp.dot(u2, w3_ref[...], preferred_element_type=jnp.float32) + b3_ref[...]
    n = jnp.sqrt(jnp.sum(v * v, axis=-1, keepdims=True))
    v_ref[0] = v / jnp.maximum(n, 1e-8)


def _mlp(base, gtab, gpos, w2bf, b2, w3bf, b3, g1, be1, g2, be2):
    L, hid = base.shape
    D = w3bf.shape[1]
    bm = min(256, L)
    grid = (5, L // bm)
    vec = lambda a: a.reshape(1, -1)
    return pl.pallas_call(
        _mlp_body,
        grid=grid,
        in_specs=[pl.BlockSpec((bm, hid), lambda g, r: (r, 0)),
                  pl.BlockSpec((1, bm, hid), lambda g, r: (g, r, 0)),
                  pl.BlockSpec((1, bm, hid), lambda g, r: (g, r, 0)),
                  pl.BlockSpec((hid, hid), lambda g, r: (0, 0)),
                  pl.BlockSpec((1, hid), lambda g, r: (0, 0)),
                  pl.BlockSpec((hid, D), lambda g, r: (0, 0)),
                  pl.BlockSpec((1, D), lambda g, r: (0, 0)),
                  pl.BlockSpec((1, hid), lambda g, r: (0, 0)),
                  pl.BlockSpec((1, hid), lambda g, r: (0, 0)),
                  pl.BlockSpec((1, hid), lambda g, r: (0, 0)),
                  pl.BlockSpec((1, hid), lambda g, r: (0, 0))],
        out_specs=pl.BlockSpec((1, bm, D), lambda g, r: (g, r, 0)),
        out_shape=jax.ShapeDtypeStruct((5, L, D), jnp.float32),
    )(base, gtab.reshape(5, L, hid), gpos.reshape(5, L, hid), w2bf, vec(b2),
      w3bf, vec(b3), vec(g1), vec(be1), vec(g2), vec(be2))


def _step_body(h_ref, ht_ref, vij_ref, vji_ref, w_ref, gd_ref, gt_ref):
    g = pl.program_id(1)
    hs = h_ref[...]
    ht = ht_ref[0]
    v1 = vij_ref[0]
    v2 = vji_ref[0]
    w = w_ref[0, :, 0:1]
    a = jnp.sum(v1 * hs, axis=-1, keepdims=True)
    b = jnp.sum(v2 * ht, axis=-1, keepdims=True)
    delta = w * (hs - 2.0 * a * v1 - ht + 2.0 * b * v2)
    c = jnp.sum(v1 * delta, axis=-1, keepdims=True)
    d = jnp.sum(v2 * delta, axis=-1, keepdims=True)
    gs = delta - 2.0 * c * v1
    gt_ref[0] = -(delta - 2.0 * d * v2)

    @pl.when(g == 0)
    def _():
        gd_ref[...] = gs

    @pl.when(g != 0)
    def _():
        gd_ref[...] += gs


def _step_dense(hcur, HT, vij, vji, w8b):
    L, D = hcur.shape
    bm = min(256, L)
    grid = (L // bm, 5)
    gdense, GT = pl.pallas_call(
        _step_body,
        grid=grid,
        in_specs=[pl.BlockSpec((bm, D), lambda r, g: (r, 0)),
                  pl.BlockSpec((1, bm, D), lambda r, g: (g, r, 0)),
                  pl.BlockSpec((1, bm, D), lambda r, g: (g, r, 0)),
                  pl.BlockSpec((1, bm, D), lambda r, g: (g, r, 0)),
                  pl.BlockSpec((1, bm, 8), lambda r, g: (g, r, 0))],
        out_specs=[pl.BlockSpec((bm, D), lambda r, g: (r, 0)),
                   pl.BlockSpec((1, bm, D), lambda r, g: (g, r, 0))],
        out_shape=[jax.ShapeDtypeStruct((L, D), jnp.float32),
                   jax.ShapeDtypeStruct((5, L, D), jnp.float32)],
    )(hcur, HT, vij, vji, w8b)
    return gdense, GT


def _upd_body(eta_ref, h_ref, h0_ref, gd_ref, gsc_ref, o_ref, *, inv_denom):
    eta = eta_ref[0]
    g = (gd_ref[...] + gsc_ref[...]) * inv_denom
    o_ref[...] = h_ref[...] - eta * (g + LAM * (h_ref[...] - h0_ref[...]))


def _update(hcur, h0, gdense, gscatter, eta_arr, inv_denom):
    L, D = hcur.shape
    bm = min(256, L)
    return pl.pallas_call(
        functools.partial(_upd_body, inv_denom=inv_denom),
        grid=(L // bm,),
        in_specs=[pl.BlockSpec(memory_space=pltpu.SMEM),
                  pl.BlockSpec((bm, D), lambda r: (r, 0)),
                  pl.BlockSpec((bm, D), lambda r: (r, 0)),
                  pl.BlockSpec((bm, D), lambda r: (r, 0)),
                  pl.BlockSpec((bm, D), lambda r: (r, 0))],
        out_specs=pl.BlockSpec((bm, D), lambda r: (r, 0)),
        out_shape=jax.ShapeDtypeStruct((L, D), jnp.float32),
    )(eta_arr, hcur, h0, gdense, gscatter)


def _gather_rows(table, idx):
    return table[idx]


def _scatter_add_rows(rows, idx, L):
    D = rows.shape[-1]
    return jnp.zeros((L, D), jnp.float32).at[idx].add(rows)


def kernel(h, attention_weights, W1, b1, g1, beta1, W2, b2, g2, beta2, W3,
           b3, eta, pos_table):
    B, L, D = h.shape
    hid = W1.shape[1]
    maxrel = (pos_table.shape[0] - 1) // 2
    h2 = h.reshape(L, D)
    attn = attention_weights.reshape(L, L)

    vals8, idx8 = _topk(attn)
    w8, iij8, iji8 = _wix(vals8, idx8, maxrel)

    t_flat = idx8.T[:5].reshape(-1)
    iij_flat = iij8.T[:5].reshape(-1)
    iji_flat = iji8.T[:5].reshape(-1)
    w8b = jnp.broadcast_to(w8.T[:5][:, :, None], (5, L, 8))

    bf = lambda a: a.astype(jnp.bfloat16)
    zb = jnp.zeros((1, hid), jnp.float32)
    A = _mm(h2, bf(W1[:D]), zb)
    Bv = _mm(h2, bf(W1[D:2 * D]), zb)
    P = _mm(pos_table, bf(W1[2 * D:]), b1.reshape(1, hid))

    Gbv = _gather_rows(Bv, t_flat)
    Ga = _gather_rows(A, t_flat)
    Pij = _gather_rows(P, iij_flat)
    Pji = _gather_rows(P, iji_flat)

    vij = _mlp(A, Gbv, Pij, bf(W2), b2, bf(W3), b3, g1, beta1, g2, beta2)
    vji = _mlp(Bv, Ga, Pji, bf(W2), b2, bf(W3), b3, g1, beta1, g2, beta2)

    n_edges = 4 * L + (L - 1)
    inv_denom = 1.0 / (n_edges * D + 1e-8)
    eta_arr = eta.reshape(1)

    hcur = h2
    for _ in range(3):
        HT = _gather_rows(hcur, t_flat).reshape(5, L, D)
        gdense, GT = _step_dense(hcur, HT, vij, vji, w8b)
        gsc = _scatter_add_rows(GT.reshape(5 * L, D), t_flat, L)
        hcur = _update(hcur, h2, gdense, gsc, eta_arr, inv_denom)
    return hcur.reshape(B, L, D)


# speedup vs baseline: 4.3830x; 1.0101x over previous
import functools

import jax
import jax.numpy as jnp
from jax import lax
from jax.experimental import pallas as pl
from jax.experimental.pallas import tpu as pltpu
from jax.experimental.pallas import tpu_sc as plsc

LAM = 0.01
NEG_INF = float("-inf")


def _topk_body(attn_ref, vals_ref, idx_ref, *, blk, L):
    r = pl.program_id(0)
    x = attn_ref[...]
    rows = r * blk + lax.broadcasted_iota(jnp.int32, (blk, L), 0)
    cols = lax.broadcasted_iota(jnp.int32, (blk, L), 1)
    cur = jnp.where(cols < rows, x, NEG_INF)
    vs, ids = [], []
    for _ in range(4):
        m = jnp.max(cur, axis=1, keepdims=True)
        cand = jnp.where(cur == m, cols, L)
        a = jnp.min(cand, axis=1, keepdims=True)
        a = jnp.where(a == L, 0, a)
        vs.append(m)
        ids.append(a)
        cur = jnp.where(cols == a, NEG_INF, cur)
    chain_v = jnp.sum(jnp.where(cols == rows - 1, x, 0.0), axis=1, keepdims=True)
    chain_t = jnp.maximum(rows[:, 0:1] - 1, 0)
    zf = jnp.zeros((blk, 1), jnp.float32)
    zi = jnp.zeros((blk, 1), jnp.int32)
    vals_ref[...] = jnp.concatenate(vs + [chain_v, zf, zf, zf], axis=1)
    idx_ref[...] = jnp.concatenate(ids + [chain_t, zi, zi, zi], axis=1)


def _topk(attn):
    L = attn.shape[0]
    blk = min(256, L)
    return pl.pallas_call(
        functools.partial(_topk_body, blk=blk, L=L),
        grid=(L // blk,),
        in_specs=[pl.BlockSpec((blk, L), lambda r: (r, 0))],
        out_specs=[pl.BlockSpec((blk, 8), lambda r: (r, 0)),
                   pl.BlockSpec((blk, 8), lambda r: (r, 0))],
        out_shape=[jax.ShapeDtypeStruct((L, 8), jnp.float32),
                   jax.ShapeDtypeStruct((L, 8), jnp.int32)],
    )(attn)


def _wix_body(vals_ref, idx_ref, w_ref, iij_ref, iji_ref, *, L, maxrel):
    vals = vals_ref[...]
    t8 = idx_ref[...]
    lane = lax.broadcasted_iota(jnp.int32, (L, 8), 1)
    rowv = lax.broadcasted_iota(jnp.int32, (L, 8), 0)
    m = jnp.max(vals, axis=0, keepdims=True)
    e = jnp.exp(vals - m)
    s = jnp.sum(e, axis=0, keepdims=True)
    wsm = e / s
    w_ref[...] = jnp.where(lane < 4, wsm, jnp.where(lane == 4, vals, 0.0))
    rel = t8 - rowv
    iij_ref[...] = jnp.clip(rel, -maxrel, maxrel) + maxrel
    iji_ref[...] = jnp.clip(-rel, -maxrel, maxrel) + maxrel


def _wix(vals8, idx8, maxrel):
    L = vals8.shape[0]
    return pl.pallas_call(
        functools.partial(_wix_body, L=L, maxrel=maxrel),
        in_specs=[pl.BlockSpec((L, 8), lambda: (0, 0))] * 2,
        out_specs=[pl.BlockSpec((L, 8), lambda: (0, 0))] * 3,
        out_shape=[jax.ShapeDtypeStruct((L, 8), jnp.float32),
                   jax.ShapeDtypeStruct((L, 8), jnp.int32),
                   jax.ShapeDtypeStruct((L, 8), jnp.int32)],
    )(vals8, idx8)


def _mm_body(x_ref, w_ref, b_ref, o_ref):
    x = x_ref[...].astype(jnp.bfloat16)
    o_ref[...] = jnp.dot(x, w_ref[...], preferred_element_type=jnp.float32) + b_ref[...]


def _mm(x, wbf, bias):
    M, K = x.shape
    N = wbf.shape[1]
    bm = min(256, M)
    return pl.pallas_call(
        _mm_body,
        grid=(pl.cdiv(M, bm),),
        in_specs=[pl.BlockSpec((bm, K), lambda r: (r, 0)),
                  pl.BlockSpec((K, N), lambda r: (0, 0)),
                  pl.BlockSpec((1, N), lambda r: (0, 0))],
        out_specs=pl.BlockSpec((bm, N), lambda r: (r, 0)),
        out_shape=jax.ShapeDtypeStruct((M, N), jnp.float32),
    )(x, wbf, bias)


def _ln(x, g, b):
    mu = jnp.mean(x, axis=-1, keepdims=True)
    v = jnp.mean((x - mu) ** 2, axis=-1, keepdims=True)
    return (x - mu) * lax.rsqrt(v + 1e-5) * g + b


def _mlp_body(base_ref, sh_ref, shp_ref, gt_ref, gp_ref, pch_ref, w2_ref,
              b2_ref, w3_ref, b3_ref, g1_ref, be1_ref, g2_ref, be2_ref,
              v_ref, *, bm):
    g = pl.program_id(0)
    base = base_ref[...]
    x1_topk = base + gt_ref[0] + gp_ref[0]
    sh_shift = jnp.concatenate([shp_ref[bm - 1:bm, :], sh_ref[:bm - 1, :]],
                               axis=0)
    x1_chain = base + sh_shift + pch_ref[...]
    x1 = jnp.where(g == 4, x1_chain, x1_topk)
    u = jax.nn.relu(_ln(x1, g1_ref[...], be1_ref[...])).astype(jnp.bfloat16)
    h2 = jnp.dot(u, w2_ref[...], preferred_element_type=jnp.float32) + b2_ref[...]
    u2 = jax.nn.relu(_ln(h2, g2_ref[...], be2_ref[...])).astype(jnp.bfloat16)
    v = jnp.dot(u2, w3_ref[...], preferred_element_type=jnp.float32) + b3_ref[...]
    n = jnp.sqrt(jnp.sum(v * v, axis=-1, keepdims=True))
    v_ref[0] = v / jnp.maximum(n, 1e-8)


def _mlp(base, sh, gtab4, gpos4, p_chain, w2bf, b2, w3bf, b3, g1, be1, g2,
         be2):
    L, hid = base.shape
    D = w3bf.shape[1]
    bm = min(256, L)
    grid = (5, L // bm)
    vec = lambda a: a.reshape(1, -1)
    g3 = lambda g: jnp.minimum(g, 3)
    return pl.pallas_call(
        functools.partial(_mlp_body, bm=bm),
        grid=grid,
        in_specs=[pl.BlockSpec((bm, hid), lambda g, r: (r, 0)),
                  pl.BlockSpec((bm, hid), lambda g, r: (r, 0)),
                  pl.BlockSpec((bm, hid), lambda g, r: (jnp.maximum(r - 1, 0), 0)),
                  pl.BlockSpec((1, bm, hid), lambda g, r: (g3(g), r, 0)),
                  pl.BlockSpec((1, bm, hid), lambda g, r: (g3(g), r, 0)),
                  pl.BlockSpec((1, hid), lambda g, r: (0, 0)),
                  pl.BlockSpec((hid, hid), lambda g, r: (0, 0)),
                  pl.BlockSpec((1, hid), lambda g, r: (0, 0)),
                  pl.BlockSpec((hid, D), lambda g, r: (0, 0)),
                  pl.BlockSpec((1, D), lambda g, r: (0, 0)),
                  pl.BlockSpec((1, hid), lambda g, r: (0, 0)),
                  pl.BlockSpec((1, hid), lambda g, r: (0, 0)),
                  pl.BlockSpec((1, hid), lambda g, r: (0, 0)),
                  pl.BlockSpec((1, hid), lambda g, r: (0, 0))],
        out_specs=pl.BlockSpec((1, bm, D), lambda g, r: (g, r, 0)),
        out_shape=jax.ShapeDtypeStruct((5, L, D), jnp.float32),
    )(base, sh, sh, gtab4.reshape(4, L, hid), gpos4.reshape(4, L, hid),
      p_chain, w2bf, vec(b2), w3bf, vec(b3), vec(g1), vec(be1), vec(g2),
      vec(be2))


def _step_body(h_ref, hp_ref, ht_ref, vij_ref, vji_ref, w_ref, gd_ref,
               gt_ref, bnd_ref, *, bm):
    g = pl.program_id(1)
    hs = h_ref[...]
    ht_chain = jnp.concatenate([hp_ref[bm - 1:bm, :], hs[:bm - 1, :]], axis=0)
    ht = jnp.where(g == 4, ht_chain, ht_ref[0])
    v1 = vij_ref[0]
    v2 = vji_ref[0]
    w = w_ref[0, :, 0:1]
    a = jnp.sum(v1 * hs, axis=-1, keepdims=True)
    b = jnp.sum(v2 * ht, axis=-1, keepdims=True)
    delta = w * (hs - 2.0 * a * v1 - ht + 2.0 * b * v2)
    c = jnp.sum(v1 * delta, axis=-1, keepdims=True)
    d = jnp.sum(v2 * delta, axis=-1, keepdims=True)
    gs = delta - 2.0 * c * v1
    gt = -(delta - 2.0 * d * v2)

    @pl.when(g < 4)
    def _():
        gt_ref[0] = gt.T

    @pl.when(g == 4)
    def _():
        bnd_ref[...] = gt[0:1].reshape(1, 1, -1)

    shup = jnp.concatenate([gt[1:], jnp.zeros_like(gt[0:1])], axis=0)
    contrib = gs + jnp.where(g == 4, shup, 0.0)

    @pl.when(g == 0)
    def _():
        gd_ref[...] = contrib

    @pl.when(g != 0)
    def _():
        gd_ref[...] += contrib


def _step_dense(hcur, HT4, vij, vji, w8b):
    L, D = hcur.shape
    bm = min(256, L)
    nb = L // bm
    g3 = lambda g: jnp.minimum(g, 3)
    return pl.pallas_call(
        functools.partial(_step_body, bm=bm),
        grid=(nb, 5),
        in_specs=[pl.BlockSpec((bm, D), lambda r, g: (r, 0)),
                  pl.BlockSpec((bm, D), lambda r, g: (jnp.maximum(r - 1, 0), 0)),
                  pl.BlockSpec((1, bm, D), lambda r, g: (g3(g), r, 0)),
                  pl.BlockSpec((1, bm, D), lambda r, g: (g, r, 0)),
                  pl.BlockSpec((1, bm, D), lambda r, g: (g, r, 0)),
                  pl.BlockSpec((1, bm, 8), lambda r, g: (g, r, 0))],
        out_specs=[pl.BlockSpec((bm, D), lambda r, g: (r, 0)),
                   pl.BlockSpec((1, D, bm), lambda r, g: (g3(g), 0, r)),
                   pl.BlockSpec((1, 1, D), lambda r, g: (r, 0, 0))],
        out_shape=[jax.ShapeDtypeStruct((L, D), jnp.float32),
                   jax.ShapeDtypeStruct((4, D, L), jnp.float32),
                   jax.ShapeDtypeStruct((nb, 1, D), jnp.float32)],
    )(hcur, hcur, HT4, vij, vji, w8b)


def _upd_body(eta_ref, h_ref, h0_ref, gd_ref, gsc_ref, bnd_ref, o_ref, *,
              inv_denom, bm, nb):
    r = pl.program_id(0)
    eta = eta_ref[0]
    bnd_next = bnd_ref[jnp.minimum(r + 1, nb - 1)]
    rowpos = lax.broadcasted_iota(jnp.int32, (bm, 1), 0)
    add = jnp.where((rowpos == bm - 1) & (r < nb - 1), bnd_next, 0.0)
    h = h_ref[...]
    g = (gd_ref[...] + gsc_ref[...].T + add) * inv_denom
    o_ref[...] = h - eta * (g + LAM * (h - h0_ref[...]))


def _update(hcur, h0, gdense, gscatter, bnd, eta_arr, inv_denom):
    L, D = hcur.shape
    bm = min(256, L)
    nb = L // bm
    return pl.pallas_call(
        functools.partial(_upd_body, inv_denom=inv_denom, bm=bm, nb=nb),
        grid=(nb,),
        in_specs=[pl.BlockSpec(memory_space=pltpu.SMEM),
                  pl.BlockSpec((bm, D), lambda r: (r, 0)),
                  pl.BlockSpec((bm, D), lambda r: (r, 0)),
                  pl.BlockSpec((bm, D), lambda r: (r, 0)),
                  pl.BlockSpec((D, bm), lambda r: (0, r)),
                  pl.BlockSpec((nb, 1, D), lambda r: (0, 0, 0))],
        out_specs=pl.BlockSpec((bm, D), lambda r: (r, 0)),
        out_shape=jax.ShapeDtypeStruct((L, D), jnp.float32),
    )(eta_arr, hcur, h0, gdense, gscatter, bnd)


_NW = 32
_CH = 64


def _gather_rows(table, idx):
    V, D = table.shape
    N = idx.shape[0]
    per_w = N // _NW
    n_chunks = per_w // _CH
    mesh = plsc.VectorSubcoreMesh(core_axis_name="c", subcore_axis_name="s")

    @functools.partial(
        pl.kernel, mesh=mesh,
        out_type=jax.ShapeDtypeStruct((N, D), jnp.float32),
        scratch_types=[pltpu.VMEM((per_w,), jnp.int32),
                       pltpu.VMEM((_CH, D), jnp.float32),
                       pltpu.VMEM((_CH, D), jnp.float32),
                       pltpu.SemaphoreType.DMA,
                       pltpu.SemaphoreType.DMA],
    )
    def k(table_hbm, idx_hbm, out_hbm, idx_v, buf0, buf1, sem0, sem1):
        wid = lax.axis_index("s") * 2 + lax.axis_index("c")
        base = pl.multiple_of(wid * per_w, per_w)
        pltpu.sync_copy(idx_hbm.at[pl.ds(base, per_w)], idx_v)
        bufs, sems = (buf0, buf1), (sem0, sem1)
        cp = pltpu.async_copy(table_hbm.at[idx_v.at[pl.ds(0, _CH)]],
                              bufs[0], sems[0])
        for ci in range(1, n_chunks):
            nxt = pltpu.async_copy(
                table_hbm.at[idx_v.at[pl.ds(ci * _CH, _CH)]],
                bufs[ci % 2], sems[ci % 2])
            cp.wait()
            pltpu.sync_copy(bufs[(ci - 1) % 2],
                            out_hbm.at[pl.ds(base + (ci - 1) * _CH, _CH)])
            cp = nxt
        cp.wait()
        pltpu.sync_copy(bufs[(n_chunks - 1) % 2],
                        out_hbm.at[pl.ds(base + (n_chunks - 1) * _CH, _CH)])

    return k(table, idx)


_SLC = 16
_SCE = 256


def _scatter_add_rows(gt4t, idx, L):
    G, D, Le = gt4t.shape
    n_slices = D // _SLC
    n_stage = Le // _SCE
    mesh = plsc.VectorSubcoreMesh(core_axis_name="c", subcore_axis_name="s")

    @functools.partial(
        pl.kernel, mesh=mesh,
        out_type=jax.ShapeDtypeStruct((D, L), jnp.float32),
        compiler_params=pltpu.CompilerParams(needs_layout_passes=False),
        scratch_types=[pltpu.VMEM((G * Le,), jnp.int32),
                       pltpu.VMEM((_SLC, _SCE), jnp.float32),
                       pltpu.VMEM((_SLC * L,), jnp.float32)],
    )
    def k(rows_hbm, idx_hbm, out_hbm, idx_v, stage, acc):
        tid = lax.axis_index("s") * 2 + lax.axis_index("c")
        pltpu.sync_copy(idx_hbm, idx_v)

        def do_slice(sl):
            c0 = pl.multiple_of(sl * _SLC, _SLC)

            def zero_body(j, carry):
                j16 = pl.multiple_of(j * 16, 16)
                acc[pl.ds(j16, 16)] = jnp.zeros((16,), jnp.float32)
                return carry

            lax.fori_loop(0, (_SLC * L) // 16, zero_body, 0)

            for g in range(G):
                def stage_body(ci, carry):
                    e0 = pl.multiple_of(ci * _SCE, _SCE)
                    pltpu.sync_copy(
                        rows_hbm.at[g, pl.ds(c0, _SLC), pl.ds(e0, _SCE)],
                        stage)

                    def sub_body(sc, carry2):
                        s16 = pl.multiple_of(sc * 16, 16)
                        t16 = idx_v[pl.ds(g * Le + e0 + s16, 16)]
                        for c in range(_SLC):
                            vals = stage[c, pl.ds(s16, 16)]
                            plsc.addupdate_scatter(acc, [c * L + t16], vals)
                        return carry2

                    lax.fori_loop(0, _SCE // 16, sub_body, 0)
                    return carry

                lax.fori_loop(0, n_stage, stage_body, 0)
            for c in range(_SLC):
                pltpu.sync_copy(acc.at[pl.ds(c * L, L)], out_hbm.at[c0 + c])

        do_slice(tid)

        @pl.when(tid < n_slices - _NW)
        def _():
            do_slice(tid + _NW)

    return k(gt4t, idx)


def kernel(h, attention_weights, W1, b1, g1, beta1, W2, b2, g2, beta2, W3,
           b3, eta, pos_table):
    B, L, D = h.shape
    hid = W1.shape[1]
    maxrel = (pos_table.shape[0] - 1) // 2
    h2 = h.reshape(L, D)
    attn = attention_weights.reshape(L, L)

    vals8, idx8 = _topk(attn)
    w8, iij8, iji8 = _wix(vals8, idx8, maxrel)

    t4 = idx8.T[:4].reshape(-1)
    iij4 = iij8.T[:4].reshape(-1)
    iji4 = iji8.T[:4].reshape(-1)
    w8b = jnp.broadcast_to(w8.T[:5][:, :, None], (5, L, 8))

    bf = lambda a: a.astype(jnp.bfloat16)
    zb = jnp.zeros((1, hid), jnp.float32)
    A = _mm(h2, bf(W1[:D]), zb)
    Bv = _mm(h2, bf(W1[D:2 * D]), zb)
    P = _mm(pos_table, bf(W1[2 * D:]), b1.reshape(1, hid))
    p_ij = P[maxrel - 1:maxrel]
    p_ji = P[maxrel + 1:maxrel + 2]

    Gbv = _gather_rows(Bv, t4)
    Ga = _gather_rows(A, t4)
    Pij = _gather_rows(P, iij4)
    Pji = _gather_rows(P, iji4)

    vij = _mlp(A, Bv, Gbv, Pij, p_ij, bf(W2), b2, bf(W3), b3, g1, beta1,
               g2, beta2)
    vji = _mlp(Bv, A, Ga, Pji, p_ji, bf(W2), b2, bf(W3), b3, g1, beta1,
               g2, beta2)

    n_edges = 4 * L + (L - 1)
    inv_denom = 1.0 / (n_edges * D + 1e-8)
    eta_arr = eta.reshape(1)

    hcur = h2
    for _ in range(3):
        HT4 = _gather_rows(hcur, t4).reshape(4, L, D)
        gdense, GT4T, bnd = _step_dense(hcur, HT4, vij, vji, w8b)
        gsc_t = _scatter_add_rows(GT4T, t4, L)
        hcur = _update(hcur, h2, gdense, gsc_t, bnd, eta_arr, inv_denom)
    return hcur.reshape(B, L, D)


# speedup vs baseline: 4.7203x; 1.0769x over previous
import functools

import jax
import jax.numpy as jnp
from jax import lax
from jax.experimental import pallas as pl
from jax.experimental.pallas import tpu as pltpu
from jax.experimental.pallas import tpu_sc as plsc

LAM = 0.01
NEG_INF = float("-inf")


def _topk_body(attn_ref, vals_ref, idx_ref, *, blk, L):
    r = pl.program_id(0)
    x = attn_ref[...]
    rows = r * blk + lax.broadcasted_iota(jnp.int32, (blk, L), 0)
    cols = lax.broadcasted_iota(jnp.int32, (blk, L), 1)
    cur = jnp.where(cols < rows, x, NEG_INF)
    vs, ids = [], []
    for _ in range(4):
        m = jnp.max(cur, axis=1, keepdims=True)
        cand = jnp.where(cur == m, cols, L)
        a = jnp.min(cand, axis=1, keepdims=True)
        a = jnp.where(a == L, 0, a)
        vs.append(m)
        ids.append(a)
        cur = jnp.where(cols == a, NEG_INF, cur)
    chain_v = jnp.sum(jnp.where(cols == rows - 1, x, 0.0), axis=1, keepdims=True)
    chain_t = jnp.maximum(rows[:, 0:1] - 1, 0)
    zf = jnp.zeros((blk, 1), jnp.float32)
    zi = jnp.zeros((blk, 1), jnp.int32)
    vals_ref[...] = jnp.concatenate(vs + [chain_v, zf, zf, zf], axis=1)
    idx_ref[...] = jnp.concatenate(ids + [chain_t, zi, zi, zi], axis=1)


def _topk(attn):
    L = attn.shape[0]
    blk = min(256, L)
    return pl.pallas_call(
        functools.partial(_topk_body, blk=blk, L=L),
        grid=(L // blk,),
        in_specs=[pl.BlockSpec((blk, L), lambda r: (r, 0))],
        out_specs=[pl.BlockSpec((blk, 8), lambda r: (r, 0)),
                   pl.BlockSpec((blk, 8), lambda r: (r, 0))],
        out_shape=[jax.ShapeDtypeStruct((L, 8), jnp.float32),
                   jax.ShapeDtypeStruct((L, 8), jnp.int32)],
    )(attn)


def _wix_body(vals_ref, idx_ref, w_ref, iij_ref, iji_ref, *, L, maxrel):
    vals = vals_ref[...]
    t8 = idx_ref[...]
    lane = lax.broadcasted_iota(jnp.int32, (L, 8), 1)
    rowv = lax.broadcasted_iota(jnp.int32, (L, 8), 0)
    m = jnp.max(vals, axis=0, keepdims=True)
    e = jnp.exp(vals - m)
    s = jnp.sum(e, axis=0, keepdims=True)
    wsm = e / s
    w_ref[...] = jnp.where(lane < 4, wsm, jnp.where(lane == 4, vals, 0.0))
    rel = t8 - rowv
    iij_ref[...] = jnp.clip(rel, -maxrel, maxrel) + maxrel
    iji_ref[...] = jnp.clip(-rel, -maxrel, maxrel) + maxrel


def _wix(vals8, idx8, maxrel):
    L = vals8.shape[0]
    return pl.pallas_call(
        functools.partial(_wix_body, L=L, maxrel=maxrel),
        in_specs=[pl.BlockSpec((L, 8), lambda: (0, 0))] * 2,
        out_specs=[pl.BlockSpec((L, 8), lambda: (0, 0))] * 3,
        out_shape=[jax.ShapeDtypeStruct((L, 8), jnp.float32),
                   jax.ShapeDtypeStruct((L, 8), jnp.int32),
                   jax.ShapeDtypeStruct((L, 8), jnp.int32)],
    )(vals8, idx8)


def _mm_body(x_ref, w_ref, b_ref, o_ref):
    x = x_ref[...].astype(jnp.bfloat16)
    o_ref[...] = jnp.dot(x, w_ref[...], preferred_element_type=jnp.float32) + b_ref[...]


def _mm(x, wbf, bias):
    M, K = x.shape
    N = wbf.shape[1]
    bm = min(256, M)
    return pl.pallas_call(
        _mm_body,
        grid=(pl.cdiv(M, bm),),
        in_specs=[pl.BlockSpec((bm, K), lambda r: (r, 0)),
                  pl.BlockSpec((K, N), lambda r: (0, 0)),
                  pl.BlockSpec((1, N), lambda r: (0, 0))],
        out_specs=pl.BlockSpec((bm, N), lambda r: (r, 0)),
        out_shape=jax.ShapeDtypeStruct((M, N), jnp.float32),
    )(x, wbf, bias)


def _ln(x, g, b):
    mu = jnp.mean(x, axis=-1, keepdims=True)
    v = jnp.mean((x - mu) ** 2, axis=-1, keepdims=True)
    return (x - mu) * lax.rsqrt(v + 1e-5) * g + b


def _mlp_body(base_ref, sh_ref, shp_ref, gt_ref, gp_ref, pch_ref, w2_ref,
              b2_ref, w3_ref, b3_ref, g1_ref, be1_ref, g2_ref, be2_ref,
              v_ref, *, bm):
    g = pl.program_id(0)
    base = base_ref[...]
    x1_topk = base + gt_ref[0] + gp_ref[0]
    sh_shift = jnp.concatenate([shp_ref[bm - 1:bm, :], sh_ref[:bm - 1, :]],
                               axis=0)
    x1_chain = base + sh_shift + pch_ref[...]
    x1 = jnp.where(g == 4, x1_chain, x1_topk)
    u = jax.nn.relu(_ln(x1, g1_ref[...], be1_ref[...])).astype(jnp.bfloat16)
    h2 = jnp.dot(u, w2_ref[...], preferred_element_type=jnp.float32) + b2_ref[...]
    u2 = jax.nn.relu(_ln(h2, g2_ref[...], be2_ref[...])).astype(jnp.bfloat16)
    v = jnp.dot(u2, w3_ref[...], preferred_element_type=jnp.float32) + b3_ref[...]
    n = jnp.sqrt(jnp.sum(v * v, axis=-1, keepdims=True))
    v_ref[0] = v / jnp.maximum(n, 1e-8)


def _mlp(base, sh, gtab4, gpos, gp_off, p_chain, w2bf, b2, w3bf, b3, g1,
         be1, g2, be2):
    L, hid = base.shape
    D = w3bf.shape[1]
    bm = min(256, L)
    grid = (5, L // bm)
    vec = lambda a: a.reshape(1, -1)
    g3 = lambda g: jnp.minimum(g, 3)
    return pl.pallas_call(
        functools.partial(_mlp_body, bm=bm),
        grid=grid,
        in_specs=[pl.BlockSpec((bm, hid), lambda g, r: (r, 0)),
                  pl.BlockSpec((bm, hid), lambda g, r: (r, 0)),
                  pl.BlockSpec((bm, hid), lambda g, r: (jnp.maximum(r - 1, 0), 0)),
                  pl.BlockSpec((1, bm, hid), lambda g, r: (g3(g), r, 0)),
                  pl.BlockSpec((1, bm, hid), lambda g, r: (g3(g) + gp_off, r, 0)),
                  pl.BlockSpec((1, hid), lambda g, r: (0, 0)),
                  pl.BlockSpec((hid, hid), lambda g, r: (0, 0)),
                  pl.BlockSpec((1, hid), lambda g, r: (0, 0)),
                  pl.BlockSpec((hid, D), lambda g, r: (0, 0)),
                  pl.BlockSpec((1, D), lambda g, r: (0, 0)),
                  pl.BlockSpec((1, hid), lambda g, r: (0, 0)),
                  pl.BlockSpec((1, hid), lambda g, r: (0, 0)),
                  pl.BlockSpec((1, hid), lambda g, r: (0, 0)),
                  pl.BlockSpec((1, hid), lambda g, r: (0, 0))],
        out_specs=pl.BlockSpec((1, bm, D), lambda g, r: (g, r, 0)),
        out_shape=jax.ShapeDtypeStruct((5, L, D), jnp.float32),
    )(base, sh, sh, gtab4.reshape(4, L, hid), gpos.reshape(-1, L, hid),
      p_chain, w2bf, vec(b2), w3bf, vec(b3), vec(g1), vec(be1), vec(g2),
      vec(be2))


def _step_body(h_ref, hp_ref, ht_ref, vij_ref, vji_ref, w_ref, gd_ref,
               gt_ref, bnd_ref, *, bm):
    g = pl.program_id(1)
    hs = h_ref[...]
    ht_chain = jnp.concatenate([hp_ref[bm - 1:bm, :], hs[:bm - 1, :]], axis=0)
    ht = jnp.where(g == 4, ht_chain, ht_ref[0])
    v1 = vij_ref[0]
    v2 = vji_ref[0]
    w = w_ref[0, :, 0:1]
    a = jnp.sum(v1 * hs, axis=-1, keepdims=True)
    b = jnp.sum(v2 * ht, axis=-1, keepdims=True)
    delta = w * (hs - 2.0 * a * v1 - ht + 2.0 * b * v2)
    c = jnp.sum(v1 * delta, axis=-1, keepdims=True)
    d = jnp.sum(v2 * delta, axis=-1, keepdims=True)
    gs = delta - 2.0 * c * v1
    gt = -(delta - 2.0 * d * v2)

    @pl.when(g < 4)
    def _():
        gt_ref[0] = gt.T

    @pl.when(g == 4)
    def _():
        bnd_ref[...] = gt[0:1].reshape(1, 1, -1)

    shup = jnp.concatenate([gt[1:], jnp.zeros_like(gt[0:1])], axis=0)
    contrib = gs + jnp.where(g == 4, shup, 0.0)

    @pl.when(g == 0)
    def _():
        gd_ref[...] = contrib

    @pl.when(g != 0)
    def _():
        gd_ref[...] += contrib


def _step_dense(hcur, HT4, vij, vji, w8b):
    L, D = hcur.shape
    bm = min(256, L)
    nb = L // bm
    g3 = lambda g: jnp.minimum(g, 3)
    return pl.pallas_call(
        functools.partial(_step_body, bm=bm),
        grid=(nb, 5),
        in_specs=[pl.BlockSpec((bm, D), lambda r, g: (r, 0)),
                  pl.BlockSpec((bm, D), lambda r, g: (jnp.maximum(r - 1, 0), 0)),
                  pl.BlockSpec((1, bm, D), lambda r, g: (g3(g), r, 0)),
                  pl.BlockSpec((1, bm, D), lambda r, g: (g, r, 0)),
                  pl.BlockSpec((1, bm, D), lambda r, g: (g, r, 0)),
                  pl.BlockSpec((1, bm, 8), lambda r, g: (g, r, 0))],
        out_specs=[pl.BlockSpec((bm, D), lambda r, g: (r, 0)),
                   pl.BlockSpec((1, D, bm), lambda r, g: (g3(g), 0, r)),
                   pl.BlockSpec((1, 1, D), lambda r, g: (r, 0, 0))],
        out_shape=[jax.ShapeDtypeStruct((L, D), jnp.float32),
                   jax.ShapeDtypeStruct((4, D, L), jnp.float32),
                   jax.ShapeDtypeStruct((nb, 1, D), jnp.float32)],
    )(hcur, hcur, HT4, vij, vji, w8b)


def _upd_body(eta_ref, h_ref, h0_ref, gd_ref, gsc_ref, bnd_ref, o_ref, *,
              inv_denom, bm, nb):
    r = pl.program_id(0)
    eta = eta_ref[0]
    bnd_next = bnd_ref[jnp.minimum(r + 1, nb - 1)]
    rowpos = lax.broadcasted_iota(jnp.int32, (bm, 1), 0)
    add = jnp.where((rowpos == bm - 1) & (r < nb - 1), bnd_next, 0.0)
    h = h_ref[...]
    g = (gd_ref[...] + gsc_ref[...].T + add) * inv_denom
    o_ref[...] = h - eta * (g + LAM * (h - h0_ref[...]))


def _update(hcur, h0, gdense, gscatter, bnd, eta_arr, inv_denom):
    L, D = hcur.shape
    bm = min(256, L)
    nb = L // bm
    return pl.pallas_call(
        functools.partial(_upd_body, inv_denom=inv_denom, bm=bm, nb=nb),
        grid=(nb,),
        in_specs=[pl.BlockSpec(memory_space=pltpu.SMEM),
                  pl.BlockSpec((bm, D), lambda r: (r, 0)),
                  pl.BlockSpec((bm, D), lambda r: (r, 0)),
                  pl.BlockSpec((bm, D), lambda r: (r, 0)),
                  pl.BlockSpec((D, bm), lambda r: (0, r)),
                  pl.BlockSpec((nb, 1, D), lambda r: (0, 0, 0))],
        out_specs=pl.BlockSpec((bm, D), lambda r: (r, 0)),
        out_shape=jax.ShapeDtypeStruct((L, D), jnp.float32),
    )(eta_arr, hcur, h0, gdense, gscatter, bnd)


_NW = 32
_CH = 32


_GNB = 4


def _gather_rows(table, idx):
    V, D = table.shape
    N = idx.shape[0]
    per_w = N // _NW
    n_chunks = per_w // _CH
    mesh = plsc.VectorSubcoreMesh(core_axis_name="c", subcore_axis_name="s")

    @functools.partial(
        pl.kernel, mesh=mesh,
        out_type=jax.ShapeDtypeStruct((N, D), jnp.float32),
        scratch_types=[pltpu.VMEM((per_w,), jnp.int32)]
        + [pltpu.VMEM((_CH, D), jnp.float32)] * _GNB
        + [pltpu.SemaphoreType.DMA] * (2 * _GNB),
    )
    def k(table_hbm, idx_hbm, out_hbm, idx_v, *bufsem):
        bufs = bufsem[:_GNB]
        gsems = bufsem[_GNB:2 * _GNB]
        wsems = bufsem[2 * _GNB:]
        wid = lax.axis_index("s") * 2 + lax.axis_index("c")
        base = pl.multiple_of(wid * per_w, per_w)
        pltpu.sync_copy(idx_hbm.at[pl.ds(base, per_w)], idx_v)
        gcp, wcp = {}, {}
        for ci in range(n_chunks):
            if ci >= _GNB:
                wcp[ci - _GNB].wait()
            gcp[ci] = pltpu.async_copy(
                table_hbm.at[idx_v.at[pl.ds(ci * _CH, _CH)]],
                bufs[ci % _GNB], gsems[ci % _GNB])
            if ci >= 1:
                gcp[ci - 1].wait()
                wcp[ci - 1] = pltpu.async_copy(
                    bufs[(ci - 1) % _GNB],
                    out_hbm.at[pl.ds(base + (ci - 1) * _CH, _CH)],
                    wsems[(ci - 1) % _GNB])
        gcp[n_chunks - 1].wait()
        wcp[n_chunks - 1] = pltpu.async_copy(
            bufs[(n_chunks - 1) % _GNB],
            out_hbm.at[pl.ds(base + (n_chunks - 1) * _CH, _CH)],
            wsems[(n_chunks - 1) % _GNB])
        for ci in range(max(0, n_chunks - _GNB), n_chunks):
            wcp[ci].wait()

    return k(table, idx)


_SLC = 16
_SCE = 256


def _scatter_add_rows(gt4t, idx, L):
    G, D, Le = gt4t.shape
    n_slices = D // _SLC
    n_stage = Le // _SCE
    mesh = plsc.VectorSubcoreMesh(core_axis_name="c", subcore_axis_name="s")

    @functools.partial(
        pl.kernel, mesh=mesh,
        out_type=jax.ShapeDtypeStruct((D, L), jnp.float32),
        compiler_params=pltpu.CompilerParams(needs_layout_passes=False),
        scratch_types=[pltpu.VMEM((G * Le,), jnp.int32),
                       pltpu.VMEM((_SLC, _SCE), jnp.float32),
                       pltpu.VMEM((_SLC * L,), jnp.float32)],
    )
    def k(rows_hbm, idx_hbm, out_hbm, idx_v, stage, acc):
        tid = lax.axis_index("s") * 2 + lax.axis_index("c")
        pltpu.sync_copy(idx_hbm, idx_v)

        def do_slice(sl):
            c0 = pl.multiple_of(sl * _SLC, _SLC)

            def zero_body(j, carry):
                j16 = pl.multiple_of(j * 16, 16)
                acc[pl.ds(j16, 16)] = jnp.zeros((16,), jnp.float32)
                return carry

            lax.fori_loop(0, (_SLC * L) // 16, zero_body, 0)

            for g in range(G):
                def stage_body(ci, carry):
                    e0 = pl.multiple_of(ci * _SCE, _SCE)
                    pltpu.sync_copy(
                        rows_hbm.at[g, pl.ds(c0, _SLC), pl.ds(e0, _SCE)],
                        stage)

                    def sub_body(sc, carry2):
                        s16 = pl.multiple_of(sc * 16, 16)
                        t16 = idx_v[pl.ds(g * Le + e0 + s16, 16)]
                        for c in range(_SLC):
                            vals = stage[c, pl.ds(s16, 16)]
                            plsc.addupdate_scatter(acc, [c * L + t16], vals)
                        return carry2

                    lax.fori_loop(0, _SCE // 16, sub_body, 0)
                    return carry

                lax.fori_loop(0, n_stage, stage_body, 0)
            for c in range(_SLC):
                pltpu.sync_copy(acc.at[pl.ds(c * L, L)], out_hbm.at[c0 + c])

        do_slice(tid)

        @pl.when(tid < n_slices - _NW)
        def _():
            do_slice(tid + _NW)

    return k(gt4t, idx)


def kernel(h, attention_weights, W1, b1, g1, beta1, W2, b2, g2, beta2, W3,
           b3, eta, pos_table):
    B, L, D = h.shape
    hid = W1.shape[1]
    maxrel = (pos_table.shape[0] - 1) // 2
    h2 = h.reshape(L, D)
    attn = attention_weights.reshape(L, L)

    vals8, idx8 = _topk(attn)
    w8, iij8, iji8 = _wix(vals8, idx8, maxrel)

    t4 = idx8.T[:4].reshape(-1)
    iij4 = iij8.T[:4].reshape(-1)
    iji4 = iji8.T[:4].reshape(-1)
    w8b = jnp.broadcast_to(w8.T[:5][:, :, None], (5, L, 8))

    bf = lambda a: a.astype(jnp.bfloat16)
    zb = jnp.zeros((1, hid), jnp.float32)
    A = _mm(h2, bf(W1[:D]), zb)
    Bv = _mm(h2, bf(W1[D:2 * D]), zb)
    P = _mm(pos_table, bf(W1[2 * D:]), b1.reshape(1, hid))
    p_ij = P[maxrel - 1:maxrel]
    p_ji = P[maxrel + 1:maxrel + 2]

    Gh = _gather_rows(h2, t4)
    Gbv = _mm(Gh, bf(W1[D:2 * D]), zb)
    Ga = _mm(Gh, bf(W1[:D]), zb)
    Pboth = _gather_rows(P, jnp.concatenate([iij4, iji4]))

    vij = _mlp(A, Bv, Gbv, Pboth, 0, p_ij, bf(W2), b2, bf(W3), b3, g1,
               beta1, g2, beta2)
    vji = _mlp(Bv, A, Ga, Pboth, 4, p_ji, bf(W2), b2, bf(W3), b3, g1,
               beta1, g2, beta2)

    n_edges = 4 * L + (L - 1)
    inv_denom = 1.0 / (n_edges * D + 1e-8)
    eta_arr = eta.reshape(1)

    hcur = h2
    for step in range(3):
        HT4 = (Gh if step == 0 else _gather_rows(hcur, t4)).reshape(4, L, D)
        gdense, GT4T, bnd = _step_dense(hcur, HT4, vij, vji, w8b)
        gsc_t = _scatter_add_rows(GT4T, t4, L)
        hcur = _update(hcur, h2, gdense, gsc_t, bnd, eta_arr, inv_denom)
    return hcur.reshape(B, L, D)


# speedup vs baseline: 5.3549x; 1.1344x over previous
import functools

import jax
import jax.numpy as jnp
from jax import lax
from jax.experimental import pallas as pl
from jax.experimental.pallas import tpu as pltpu
from jax.experimental.pallas import tpu_sc as plsc

LAM = 0.01
NEG_INF = float("-inf")


def _topk_body(attn_ref, vals_ref, idx_ref, *, blk, L):
    r = pl.program_id(0)
    x = attn_ref[...]
    rows = r * blk + lax.broadcasted_iota(jnp.int32, (blk, L), 0)
    cols = lax.broadcasted_iota(jnp.int32, (blk, L), 1)
    cur = jnp.where(cols < rows, x, NEG_INF)
    vs, ids = [], []
    for _ in range(4):
        m = jnp.max(cur, axis=1, keepdims=True)
        cand = jnp.where(cur == m, cols, L)
        a = jnp.min(cand, axis=1, keepdims=True)
        a = jnp.where(a == L, 0, a)
        vs.append(m)
        ids.append(a)
        cur = jnp.where(cols == a, NEG_INF, cur)
    chain_v = jnp.sum(jnp.where(cols == rows - 1, x, 0.0), axis=1, keepdims=True)
    chain_t = jnp.maximum(rows[:, 0:1] - 1, 0)
    zf = jnp.zeros((blk, 1), jnp.float32)
    zi = jnp.zeros((blk, 1), jnp.int32)
    vals_ref[...] = jnp.concatenate(vs + [chain_v, zf, zf, zf], axis=1)
    idx_ref[...] = jnp.concatenate(ids + [chain_t, zi, zi, zi], axis=1)


def _topk(attn):
    L = attn.shape[0]
    blk = min(256, L)
    return pl.pallas_call(
        functools.partial(_topk_body, blk=blk, L=L),
        grid=(L // blk,),
        in_specs=[pl.BlockSpec((blk, L), lambda r: (r, 0))],
        out_specs=[pl.BlockSpec((blk, 8), lambda r: (r, 0)),
                   pl.BlockSpec((blk, 8), lambda r: (r, 0))],
        out_shape=[jax.ShapeDtypeStruct((L, 8), jnp.float32),
                   jax.ShapeDtypeStruct((L, 8), jnp.int32)],
    )(attn)


def _wix_body(vals_ref, idx_ref, w_ref, iij_ref, iji_ref, *, L, maxrel):
    vals = vals_ref[...]
    t8 = idx_ref[...]
    lane = lax.broadcasted_iota(jnp.int32, (L, 8), 1)
    rowv = lax.broadcasted_iota(jnp.int32, (L, 8), 0)
    m = jnp.max(vals, axis=0, keepdims=True)
    e = jnp.exp(vals - m)
    s = jnp.sum(e, axis=0, keepdims=True)
    wsm = e / s
    w_ref[...] = jnp.where(lane < 4, wsm, jnp.where(lane == 4, vals, 0.0))
    rel = t8 - rowv
    iij_ref[...] = jnp.clip(rel, -maxrel, maxrel) + maxrel
    iji_ref[...] = jnp.clip(-rel, -maxrel, maxrel) + maxrel


def _wix(vals8, idx8, maxrel):
    L = vals8.shape[0]
    return pl.pallas_call(
        functools.partial(_wix_body, L=L, maxrel=maxrel),
        in_specs=[pl.BlockSpec((L, 8), lambda: (0, 0))] * 2,
        out_specs=[pl.BlockSpec((L, 8), lambda: (0, 0))] * 3,
        out_shape=[jax.ShapeDtypeStruct((L, 8), jnp.float32),
                   jax.ShapeDtypeStruct((L, 8), jnp.int32),
                   jax.ShapeDtypeStruct((L, 8), jnp.int32)],
    )(vals8, idx8)


def _mm_body(x_ref, w_ref, b_ref, o_ref):
    x = x_ref[...].astype(jnp.bfloat16)
    o_ref[...] = jnp.dot(x, w_ref[...], preferred_element_type=jnp.float32) + b_ref[...]


def _mm(x, wbf, bias):
    M, K = x.shape
    N = wbf.shape[1]
    bm = min(256, M)
    return pl.pallas_call(
        _mm_body,
        grid=(pl.cdiv(M, bm),),
        in_specs=[pl.BlockSpec((bm, K), lambda r: (r, 0)),
                  pl.BlockSpec((K, N), lambda r: (0, 0)),
                  pl.BlockSpec((1, N), lambda r: (0, 0))],
        out_specs=pl.BlockSpec((bm, N), lambda r: (r, 0)),
        out_shape=jax.ShapeDtypeStruct((M, N), jnp.float32),
    )(x, wbf, bias)


def _ln(x, g, b):
    mu = jnp.mean(x, axis=-1, keepdims=True)
    v = jnp.mean((x - mu) ** 2, axis=-1, keepdims=True)
    return (x - mu) * lax.rsqrt(v + 1e-5) * g + b


def _mlp_body(base_ref, sh_ref, shp_ref, gt_ref, gp_ref, pch_ref, w2_ref,
              b2_ref, w3_ref, b3_ref, g1_ref, be1_ref, g2_ref, be2_ref,
              v_ref, *, bm):
    g = pl.program_id(0)
    base = base_ref[...]
    x1_topk = base + gt_ref[0] + gp_ref[0]
    sh_shift = jnp.concatenate([shp_ref[bm - 1:bm, :], sh_ref[:bm - 1, :]],
                               axis=0)
    x1_chain = base + sh_shift + pch_ref[...]
    x1 = jnp.where(g == 4, x1_chain, x1_topk)
    u = jax.nn.relu(_ln(x1, g1_ref[...], be1_ref[...])).astype(jnp.bfloat16)
    h2 = jnp.dot(u, w2_ref[...], preferred_element_type=jnp.float32) + b2_ref[...]
    u2 = jax.nn.relu(_ln(h2, g2_ref[...], be2_ref[...])).astype(jnp.bfloat16)
    v = jnp.dot(u2, w3_ref[...], preferred_element_type=jnp.float32) + b3_ref[...]
    n = jnp.sqrt(jnp.sum(v * v, axis=-1, keepdims=True))
    v_ref[0] = v / jnp.maximum(n, 1e-8)


def _mlp(base, sh, gtab4, gpos, gp_off, p_chain, w2bf, b2, w3bf, b3, g1,
         be1, g2, be2):
    L, hid = base.shape
    D = w3bf.shape[1]
    bm = min(256, L)
    grid = (5, L // bm)
    vec = lambda a: a.reshape(1, -1)
    g3 = lambda g: jnp.minimum(g, 3)
    return pl.pallas_call(
        functools.partial(_mlp_body, bm=bm),
        grid=grid,
        in_specs=[pl.BlockSpec((bm, hid), lambda g, r: (r, 0)),
                  pl.BlockSpec((bm, hid), lambda g, r: (r, 0)),
                  pl.BlockSpec((bm, hid), lambda g, r: (jnp.maximum(r - 1, 0), 0)),
                  pl.BlockSpec((1, bm, hid), lambda g, r: (g3(g), r, 0)),
                  pl.BlockSpec((1, bm, hid), lambda g, r: (g3(g) + gp_off, r, 0)),
                  pl.BlockSpec((1, hid), lambda g, r: (0, 0)),
                  pl.BlockSpec((hid, hid), lambda g, r: (0, 0)),
                  pl.BlockSpec((1, hid), lambda g, r: (0, 0)),
                  pl.BlockSpec((hid, D), lambda g, r: (0, 0)),
                  pl.BlockSpec((1, D), lambda g, r: (0, 0)),
                  pl.BlockSpec((1, hid), lambda g, r: (0, 0)),
                  pl.BlockSpec((1, hid), lambda g, r: (0, 0)),
                  pl.BlockSpec((1, hid), lambda g, r: (0, 0)),
                  pl.BlockSpec((1, hid), lambda g, r: (0, 0))],
        out_specs=pl.BlockSpec((1, bm, D), lambda g, r: (g, r, 0)),
        out_shape=jax.ShapeDtypeStruct((5, L, D), jnp.float32),
    )(base, sh, sh, gtab4.reshape(4, L, hid), gpos.reshape(-1, L, hid),
      p_chain, w2bf, vec(b2), w3bf, vec(b3), vec(g1), vec(be1), vec(g2),
      vec(be2))


def _step_body(h_ref, hp_ref, ht_ref, vij_ref, vji_ref, w_ref, gd_ref,
               gt_ref, bnd_ref, *, bm):
    g = pl.program_id(1)
    hs = h_ref[...]
    ht_chain = jnp.concatenate([hp_ref[bm - 1:bm, :], hs[:bm - 1, :]], axis=0)
    ht = jnp.where(g == 4, ht_chain, ht_ref[0])
    v1 = vij_ref[0]
    v2 = vji_ref[0]
    w = w_ref[0, :, 0:1]
    a = jnp.sum(v1 * hs, axis=-1, keepdims=True)
    b = jnp.sum(v2 * ht, axis=-1, keepdims=True)
    delta = w * (hs - 2.0 * a * v1 - ht + 2.0 * b * v2)
    c = jnp.sum(v1 * delta, axis=-1, keepdims=True)
    d = jnp.sum(v2 * delta, axis=-1, keepdims=True)
    gs = delta - 2.0 * c * v1
    gt = -(delta - 2.0 * d * v2)

    @pl.when(g < 4)
    def _():
        gt_ref[0] = gt.T

    @pl.when(g == 4)
    def _():
        bnd_ref[...] = gt[0:1].reshape(1, 1, -1)

    shup = jnp.concatenate([gt[1:], jnp.zeros_like(gt[0:1])], axis=0)
    contrib = gs + jnp.where(g == 4, shup, 0.0)

    @pl.when(g == 0)
    def _():
        gd_ref[...] = contrib

    @pl.when(g != 0)
    def _():
        gd_ref[...] += contrib


def _step_dense(hcur, HT4, vij, vji, w8b):
    L, D = hcur.shape
    bm = min(256, L)
    nb = L // bm
    g3 = lambda g: jnp.minimum(g, 3)
    return pl.pallas_call(
        functools.partial(_step_body, bm=bm),
        grid=(nb, 5),
        in_specs=[pl.BlockSpec((bm, D), lambda r, g: (r, 0)),
                  pl.BlockSpec((bm, D), lambda r, g: (jnp.maximum(r - 1, 0), 0)),
                  pl.BlockSpec((1, bm, D), lambda r, g: (g3(g), r, 0)),
                  pl.BlockSpec((1, bm, D), lambda r, g: (g, r, 0)),
                  pl.BlockSpec((1, bm, D), lambda r, g: (g, r, 0)),
                  pl.BlockSpec((1, bm, 8), lambda r, g: (g, r, 0))],
        out_specs=[pl.BlockSpec((bm, D), lambda r, g: (r, 0)),
                   pl.BlockSpec((1, D, bm), lambda r, g: (g3(g), 0, r)),
                   pl.BlockSpec((1, 1, D), lambda r, g: (r, 0, 0))],
        out_shape=[jax.ShapeDtypeStruct((L, D), jnp.float32),
                   jax.ShapeDtypeStruct((4, D, L), jnp.float32),
                   jax.ShapeDtypeStruct((nb, 1, D), jnp.float32)],
    )(hcur, hcur, HT4, vij, vji, w8b)


def _upd_body(eta_ref, h_ref, h0_ref, gd_ref, gsc_ref, bnd_ref, o_ref, *,
              inv_denom, bm, nb):
    r = pl.program_id(0)
    eta = eta_ref[0]
    bnd_next = bnd_ref[jnp.minimum(r + 1, nb - 1)]
    rowpos = lax.broadcasted_iota(jnp.int32, (bm, 1), 0)
    add = jnp.where((rowpos == bm - 1) & (r < nb - 1), bnd_next, 0.0)
    h = h_ref[...]
    g = (gd_ref[...] + gsc_ref[...].T + add) * inv_denom
    o_ref[...] = h - eta * (g + LAM * (h - h0_ref[...]))


def _update(hcur, h0, gdense, gscatter, bnd, eta_arr, inv_denom):
    L, D = hcur.shape
    bm = min(256, L)
    nb = L // bm
    return pl.pallas_call(
        functools.partial(_upd_body, inv_denom=inv_denom, bm=bm, nb=nb),
        grid=(nb,),
        in_specs=[pl.BlockSpec(memory_space=pltpu.SMEM),
                  pl.BlockSpec((bm, D), lambda r: (r, 0)),
                  pl.BlockSpec((bm, D), lambda r: (r, 0)),
                  pl.BlockSpec((bm, D), lambda r: (r, 0)),
                  pl.BlockSpec((D, bm), lambda r: (0, r)),
                  pl.BlockSpec((nb, 1, D), lambda r: (0, 0, 0))],
        out_specs=pl.BlockSpec((bm, D), lambda r: (r, 0)),
        out_shape=jax.ShapeDtypeStruct((L, D), jnp.float32),
    )(eta_arr, hcur, h0, gdense, gscatter, bnd)


_NW = 32
_CH = 32


_GNB = 4


def _gather_rows(table, idx):
    V, D = table.shape
    N = idx.shape[0]
    per_w = N // _NW
    n_chunks = per_w // _CH
    mesh = plsc.VectorSubcoreMesh(core_axis_name="c", subcore_axis_name="s")

    @functools.partial(
        pl.kernel, mesh=mesh,
        out_type=jax.ShapeDtypeStruct((N, D), jnp.float32),
        scratch_types=[pltpu.VMEM((per_w,), jnp.int32)]
        + [pltpu.VMEM((_CH, D), jnp.float32)] * _GNB
        + [pltpu.SemaphoreType.DMA] * (2 * _GNB),
    )
    def k(table_hbm, idx_hbm, out_hbm, idx_v, *bufsem):
        bufs = bufsem[:_GNB]
        gsems = bufsem[_GNB:2 * _GNB]
        wsems = bufsem[2 * _GNB:]
        wid = lax.axis_index("s") * 2 + lax.axis_index("c")
        base = pl.multiple_of(wid * per_w, per_w)
        pltpu.sync_copy(idx_hbm.at[pl.ds(base, per_w)], idx_v)
        gcp, wcp = {}, {}
        for ci in range(n_chunks):
            if ci >= _GNB:
                wcp[ci - _GNB].wait()
            gcp[ci] = pltpu.async_copy(
                table_hbm.at[idx_v.at[pl.ds(ci * _CH, _CH)]],
                bufs[ci % _GNB], gsems[ci % _GNB])
            if ci >= 1:
                gcp[ci - 1].wait()
                wcp[ci - 1] = pltpu.async_copy(
                    bufs[(ci - 1) % _GNB],
                    out_hbm.at[pl.ds(base + (ci - 1) * _CH, _CH)],
                    wsems[(ci - 1) % _GNB])
        gcp[n_chunks - 1].wait()
        wcp[n_chunks - 1] = pltpu.async_copy(
            bufs[(n_chunks - 1) % _GNB],
            out_hbm.at[pl.ds(base + (n_chunks - 1) * _CH, _CH)],
            wsems[(n_chunks - 1) % _GNB])
        for ci in range(max(0, n_chunks - _GNB), n_chunks):
            wcp[ci].wait()

    return k(table, idx)


_SLC = 16
_SCE = 256


def _scatter_add_rows(gt4t, idx, L):
    G, D, Le = gt4t.shape
    n_slices = D // _SLC
    n_stage = Le // _SCE
    n_chunks = G * n_stage
    mesh = plsc.VectorSubcoreMesh(core_axis_name="c", subcore_axis_name="s")

    @functools.partial(
        pl.kernel, mesh=mesh,
        out_type=jax.ShapeDtypeStruct((D, L), jnp.float32),
        compiler_params=pltpu.CompilerParams(needs_layout_passes=False),
        scratch_types=[pltpu.VMEM((G * Le,), jnp.int32),
                       pltpu.VMEM((_SLC, _SCE), jnp.float32),
                       pltpu.VMEM((_SLC, _SCE), jnp.float32),
                       pltpu.VMEM((_SLC * L,), jnp.float32)]
        + [pltpu.SemaphoreType.DMA] * 6,
    )
    def k(rows_hbm, idx_hbm, out_hbm, idx_v, st0, st1, acc, s0, s1, *wsems):
        tid = lax.axis_index("s") * 2 + lax.axis_index("c")
        pltpu.sync_copy(idx_hbm, idx_v)
        stages = (st0, st1)
        ssems = (s0, s1)

        def do_slice(sl):
            c0 = pl.multiple_of(sl * _SLC, _SLC)

            def zero_body(j, carry):
                j16 = pl.multiple_of(j * 16, 16)
                acc[pl.ds(j16, 16)] = jnp.zeros((16,), jnp.float32)
                return carry

            lax.fori_loop(0, (_SLC * L) // 16, zero_body, 0)

            def start(ci, b):
                g = ci // n_stage
                e0 = (ci % n_stage) * _SCE
                pltpu.async_copy(
                    rows_hbm.at[g, pl.ds(c0, _SLC), pl.ds(e0, _SCE)],
                    stages[b], ssems[b])

            def process(ci, b):
                g = ci // n_stage
                e0 = (ci % n_stage) * _SCE
                stage = stages[b]
                off = pl.multiple_of(g * Le + e0, _SCE)

                def sub_body(sc, carry2):
                    s16 = pl.multiple_of(sc * 16, 16)
                    t16 = idx_v[pl.ds(off + s16, 16)]
                    for c in range(_SLC):
                        vals = stage[c, pl.ds(s16, 16)]
                        plsc.addupdate_scatter(acc, [c * L + t16], vals)
                    return carry2

                lax.fori_loop(0, _SCE // 16, sub_body, 0)

            start(0, 0)
            start(1, 1)

            def ring_body(it, carry):
                base = it * 2
                for b in range(2):
                    ci = base + b
                    pltpu.make_async_copy(
                        rows_hbm.at[0, pl.ds(0, _SLC), pl.ds(0, _SCE)],
                        stages[b], ssems[b]).wait()
                    process(ci, b)

                    @pl.when(ci + 2 < n_chunks)
                    def _():
                        start(ci + 2, b)
                return carry

            lax.fori_loop(0, n_chunks // 2, ring_body, 0)

            wcps = []
            for c in range(_SLC):
                if c >= 4:
                    wcps[c - 4].wait()
                wcps.append(pltpu.async_copy(
                    acc.at[pl.ds(c * L, L)], out_hbm.at[c0 + c],
                    wsems[c % 4]))
            for c in range(_SLC - 4, _SLC):
                wcps[c].wait()

        do_slice(tid)

        @pl.when(tid < n_slices - _NW)
        def _():
            do_slice(tid + _NW)

    return k(gt4t, idx)


def kernel(h, attention_weights, W1, b1, g1, beta1, W2, b2, g2, beta2, W3,
           b3, eta, pos_table):
    B, L, D = h.shape
    hid = W1.shape[1]
    maxrel = (pos_table.shape[0] - 1) // 2
    h2 = h.reshape(L, D)
    attn = attention_weights.reshape(L, L)

    vals8, idx8 = _topk(attn)
    w8, iij8, iji8 = _wix(vals8, idx8, maxrel)

    t4 = idx8.T[:4].reshape(-1)
    iij4 = iij8.T[:4].reshape(-1)
    iji4 = iji8.T[:4].reshape(-1)
    w8b = jnp.broadcast_to(w8.T[:5][:, :, None], (5, L, 8))

    bf = lambda a: a.astype(jnp.bfloat16)
    zb = jnp.zeros((1, hid), jnp.float32)
    A = _mm(h2, bf(W1[:D]), zb)
    Bv = _mm(h2, bf(W1[D:2 * D]), zb)
    P = _mm(pos_table, bf(W1[2 * D:]), b1.reshape(1, hid))
    p_ij = P[maxrel - 1:maxrel]
    p_ji = P[maxrel + 1:maxrel + 2]

    Gh = _gather_rows(h2, t4)
    Gbv = _mm(Gh, bf(W1[D:2 * D]), zb)
    Ga = _mm(Gh, bf(W1[:D]), zb)
    Pboth = _gather_rows(P, jnp.concatenate([iij4, iji4]))

    vij = _mlp(A, Bv, Gbv, Pboth, 0, p_ij, bf(W2), b2, bf(W3), b3, g1,
               beta1, g2, beta2)
    vji = _mlp(Bv, A, Ga, Pboth, 4, p_ji, bf(W2), b2, bf(W3), b3, g1,
               beta1, g2, beta2)

    n_edges = 4 * L + (L - 1)
    inv_denom = 1.0 / (n_edges * D + 1e-8)
    eta_arr = eta.reshape(1)

    hcur = h2
    for step in range(3):
        HT4 = (Gh if step == 0 else _gather_rows(hcur, t4)).reshape(4, L, D)
        gdense, GT4T, bnd = _step_dense(hcur, HT4, vij, vji, w8b)
        gsc_t = _scatter_add_rows(GT4T, t4, L)
        hcur = _update(hcur, h2, gdense, gsc_t, bnd, eta_arr, inv_denom)
    return hcur.reshape(B, L, D)


# speedup vs baseline: 5.9311x; 1.1076x over previous
import functools

import jax
import jax.numpy as jnp
from jax import lax
from jax.experimental import pallas as pl
from jax.experimental.pallas import tpu as pltpu
from jax.experimental.pallas import tpu_sc as plsc

LAM = 0.01
NEG_INF = float("-inf")


def _topk_body(attn_ref, vals_ref, idx_ref, *, blk, L):
    r = pl.program_id(0)
    x = attn_ref[...]
    rows = r * blk + lax.broadcasted_iota(jnp.int32, (blk, L), 0)
    cols = lax.broadcasted_iota(jnp.int32, (blk, L), 1)
    cur = jnp.where(cols < rows, x, NEG_INF)
    vs, ids = [], []
    for _ in range(4):
        m = jnp.max(cur, axis=1, keepdims=True)
        cand = jnp.where(cur == m, cols, L)
        a = jnp.min(cand, axis=1, keepdims=True)
        a = jnp.where(a == L, 0, a)
        vs.append(m)
        ids.append(a)
        cur = jnp.where(cols == a, NEG_INF, cur)
    chain_v = jnp.sum(jnp.where(cols == rows - 1, x, 0.0), axis=1, keepdims=True)
    chain_t = jnp.maximum(rows[:, 0:1] - 1, 0)
    zf = jnp.zeros((blk, 1), jnp.float32)
    zi = jnp.zeros((blk, 1), jnp.int32)
    vals_ref[...] = jnp.concatenate(vs + [chain_v, zf, zf, zf], axis=1)
    idx_ref[...] = jnp.concatenate(ids + [chain_t, zi, zi, zi], axis=1)


def _topk(attn):
    L = attn.shape[0]
    blk = min(256, L)
    return pl.pallas_call(
        functools.partial(_topk_body, blk=blk, L=L),
        grid=(L // blk,),
        in_specs=[pl.BlockSpec((blk, L), lambda r: (r, 0))],
        out_specs=[pl.BlockSpec((blk, 8), lambda r: (r, 0)),
                   pl.BlockSpec((blk, 8), lambda r: (r, 0))],
        out_shape=[jax.ShapeDtypeStruct((L, 8), jnp.float32),
                   jax.ShapeDtypeStruct((L, 8), jnp.int32)],
    )(attn)


def _wix_body(vals_ref, idx_ref, w_ref, iij_ref, iji_ref, *, L, maxrel):
    vals = vals_ref[...]
    t8 = idx_ref[...]
    lane = lax.broadcasted_iota(jnp.int32, (L, 8), 1)
    rowv = lax.broadcasted_iota(jnp.int32, (L, 8), 0)
    m = jnp.max(vals, axis=0, keepdims=True)
    e = jnp.exp(vals - m)
    s = jnp.sum(e, axis=0, keepdims=True)
    wsm = e / s
    w_ref[...] = jnp.where(lane < 4, wsm, jnp.where(lane == 4, vals, 0.0))
    rel = t8 - rowv
    iij_ref[...] = jnp.clip(rel, -maxrel, maxrel) + maxrel
    iji_ref[...] = jnp.clip(-rel, -maxrel, maxrel) - 1


def _wix(vals8, idx8, maxrel):
    L = vals8.shape[0]
    return pl.pallas_call(
        functools.partial(_wix_body, L=L, maxrel=maxrel),
        in_specs=[pl.BlockSpec((L, 8), lambda: (0, 0))] * 2,
        out_specs=[pl.BlockSpec((L, 8), lambda: (0, 0))] * 3,
        out_shape=[jax.ShapeDtypeStruct((L, 8), jnp.float32),
                   jax.ShapeDtypeStruct((L, 8), jnp.int32),
                   jax.ShapeDtypeStruct((L, 8), jnp.int32)],
    )(vals8, idx8)


def _mm_body(x_ref, w_ref, b_ref, o_ref):
    x = x_ref[...].astype(jnp.bfloat16)
    o_ref[...] = jnp.dot(x, w_ref[...], preferred_element_type=jnp.float32) + b_ref[...]


def _mm(x, wbf, bias):
    M, K = x.shape
    N = wbf.shape[1]
    bm = min(256, M)
    return pl.pallas_call(
        _mm_body,
        grid=(pl.cdiv(M, bm),),
        in_specs=[pl.BlockSpec((bm, K), lambda r: (r, 0)),
                  pl.BlockSpec((K, N), lambda r: (0, 0)),
                  pl.BlockSpec((1, N), lambda r: (0, 0))],
        out_specs=pl.BlockSpec((bm, N), lambda r: (r, 0)),
        out_shape=jax.ShapeDtypeStruct((M, N), jnp.float32),
    )(x, wbf, bias)


def _ln(x, g, b):
    mu = jnp.mean(x, axis=-1, keepdims=True)
    v = jnp.mean((x - mu) ** 2, axis=-1, keepdims=True)
    return (x - mu) * lax.rsqrt(v + 1e-5) * g + b


def _mlp_body(base_ref, sh_ref, shp_ref, gt_ref, idx_ref, phi_ref, plo_ref,
              pch_ref, w2_ref, b2_ref, w3_ref, b3_ref, g1_ref, be1_ref,
              g2_ref, be2_ref, v_ref, *, bm, maxrel):
    g = pl.program_id(0)
    base = base_ref[...]
    lane8 = lax.broadcasted_iota(jnp.int32, (bm, 8), 1)
    sel = jnp.sum(jnp.where(lane8 == jnp.minimum(g, 3), idx_ref[...], 0),
                  axis=1)
    cols = lax.broadcasted_iota(jnp.int32, (bm, maxrel), 1)
    oh = (cols == sel[:, None]).astype(jnp.bfloat16)
    pe = (jnp.dot(oh, phi_ref[...], preferred_element_type=jnp.float32)
          + jnp.dot(oh, plo_ref[...], preferred_element_type=jnp.float32))
    x1_topk = base + gt_ref[0] + pe
    sh_shift = jnp.concatenate([shp_ref[bm - 1:bm, :], sh_ref[:bm - 1, :]],
                               axis=0)
    x1_chain = base + sh_shift + pch_ref[...]
    x1 = jnp.where(g == 4, x1_chain, x1_topk)
    u = jax.nn.relu(_ln(x1, g1_ref[...], be1_ref[...])).astype(jnp.bfloat16)
    h2 = jnp.dot(u, w2_ref[...], preferred_element_type=jnp.float32) + b2_ref[...]
    u2 = jax.nn.relu(_ln(h2, g2_ref[...], be2_ref[...])).astype(jnp.bfloat16)
    v = jnp.dot(u2, w3_ref[...], preferred_element_type=jnp.float32) + b3_ref[...]
    n = jnp.sqrt(jnp.sum(v * v, axis=-1, keepdims=True))
    v_ref[0] = v / jnp.maximum(n, 1e-8)


def _mlp(base, sh, gtab4, idx8, phi, plo, p_chain, w2bf, b2, w3bf, b3, g1,
         be1, g2, be2):
    L, hid = base.shape
    D = w3bf.shape[1]
    maxrel = phi.shape[0]
    bm = min(256, L)
    grid = (5, L // bm)
    vec = lambda a: a.reshape(1, -1)
    g3 = lambda g: jnp.minimum(g, 3)
    return pl.pallas_call(
        functools.partial(_mlp_body, bm=bm, maxrel=maxrel),
        grid=grid,
        in_specs=[pl.BlockSpec((bm, hid), lambda g, r: (r, 0)),
                  pl.BlockSpec((bm, hid), lambda g, r: (r, 0)),
                  pl.BlockSpec((bm, hid), lambda g, r: (jnp.maximum(r - 1, 0), 0)),
                  pl.BlockSpec((1, bm, hid), lambda g, r: (g3(g), r, 0)),
                  pl.BlockSpec((bm, 8), lambda g, r: (r, 0)),
                  pl.BlockSpec((maxrel, hid), lambda g, r: (0, 0)),
                  pl.BlockSpec((maxrel, hid), lambda g, r: (0, 0)),
                  pl.BlockSpec((1, hid), lambda g, r: (0, 0)),
                  pl.BlockSpec((hid, hid), lambda g, r: (0, 0)),
                  pl.BlockSpec((1, hid), lambda g, r: (0, 0)),
                  pl.BlockSpec((hid, D), lambda g, r: (0, 0)),
                  pl.BlockSpec((1, D), lambda g, r: (0, 0)),
                  pl.BlockSpec((1, hid), lambda g, r: (0, 0)),
                  pl.BlockSpec((1, hid), lambda g, r: (0, 0)),
                  pl.BlockSpec((1, hid), lambda g, r: (0, 0)),
                  pl.BlockSpec((1, hid), lambda g, r: (0, 0))],
        out_specs=pl.BlockSpec((1, bm, D), lambda g, r: (g, r, 0)),
        out_shape=jax.ShapeDtypeStruct((5, L, D), jnp.float32),
    )(base, sh, sh, gtab4.reshape(4, L, hid), idx8, phi, plo,
      p_chain, w2bf, vec(b2), w3bf, vec(b3), vec(g1), vec(be1), vec(g2),
      vec(be2))


def _step_body(h_ref, hp_ref, ht_ref, vij_ref, vji_ref, w_ref, gd_ref,
               gt_ref, bnd_ref, *, bm):
    g = pl.program_id(1)
    hs = h_ref[...]
    ht_chain = jnp.concatenate([hp_ref[bm - 1:bm, :], hs[:bm - 1, :]], axis=0)
    ht = jnp.where(g == 4, ht_chain, ht_ref[0])
    v1 = vij_ref[0]
    v2 = vji_ref[0]
    w = w_ref[0, :, 0:1]
    a = jnp.sum(v1 * hs, axis=-1, keepdims=True)
    b = jnp.sum(v2 * ht, axis=-1, keepdims=True)
    delta = w * (hs - 2.0 * a * v1 - ht + 2.0 * b * v2)
    c = jnp.sum(v1 * delta, axis=-1, keepdims=True)
    d = jnp.sum(v2 * delta, axis=-1, keepdims=True)
    gs = delta - 2.0 * c * v1
    gt = -(delta - 2.0 * d * v2)

    @pl.when(g < 4)
    def _():
        gt_ref[0] = gt.T

    @pl.when(g == 4)
    def _():
        bnd_ref[...] = gt[0:1].reshape(1, 1, -1)

    shup = jnp.concatenate([gt[1:], jnp.zeros_like(gt[0:1])], axis=0)
    contrib = gs + jnp.where(g == 4, shup, 0.0)

    @pl.when(g == 0)
    def _():
        gd_ref[...] = contrib

    @pl.when(g != 0)
    def _():
        gd_ref[...] += contrib


def _step_dense(hcur, HT4, vij, vji, w8b):
    L, D = hcur.shape
    bm = min(256, L)
    nb = L // bm
    g3 = lambda g: jnp.minimum(g, 3)
    return pl.pallas_call(
        functools.partial(_step_body, bm=bm),
        grid=(nb, 5),
        in_specs=[pl.BlockSpec((bm, D), lambda r, g: (r, 0)),
                  pl.BlockSpec((bm, D), lambda r, g: (jnp.maximum(r - 1, 0), 0)),
                  pl.BlockSpec((1, bm, D), lambda r, g: (g3(g), r, 0)),
                  pl.BlockSpec((1, bm, D), lambda r, g: (g, r, 0)),
                  pl.BlockSpec((1, bm, D), lambda r, g: (g, r, 0)),
                  pl.BlockSpec((1, bm, 8), lambda r, g: (g, r, 0))],
        out_specs=[pl.BlockSpec((bm, D), lambda r, g: (r, 0)),
                   pl.BlockSpec((1, D, bm), lambda r, g: (g3(g), 0, r)),
                   pl.BlockSpec((1, 1, D), lambda r, g: (r, 0, 0))],
        out_shape=[jax.ShapeDtypeStruct((L, D), jnp.float32),
                   jax.ShapeDtypeStruct((4, D, L), jnp.float32),
                   jax.ShapeDtypeStruct((nb, 1, D), jnp.float32)],
    )(hcur, hcur, HT4, vij, vji, w8b)


def _upd_body(eta_ref, h_ref, h0_ref, gd_ref, gsc_ref, bnd_ref, o_ref, *,
              inv_denom, bm, nb):
    r = pl.program_id(0)
    eta = eta_ref[0]
    bnd_next = bnd_ref[jnp.minimum(r + 1, nb - 1)]
    rowpos = lax.broadcasted_iota(jnp.int32, (bm, 1), 0)
    add = jnp.where((rowpos == bm - 1) & (r < nb - 1), bnd_next, 0.0)
    h = h_ref[...]
    g = (gd_ref[...] + gsc_ref[...].T + add) * inv_denom
    o_ref[...] = h - eta * (g + LAM * (h - h0_ref[...]))


def _update(hcur, h0, gdense, gscatter, bnd, eta_arr, inv_denom):
    L, D = hcur.shape
    bm = min(256, L)
    nb = L // bm
    return pl.pallas_call(
        functools.partial(_upd_body, inv_denom=inv_denom, bm=bm, nb=nb),
        grid=(nb,),
        in_specs=[pl.BlockSpec(memory_space=pltpu.SMEM),
                  pl.BlockSpec((bm, D), lambda r: (r, 0)),
                  pl.BlockSpec((bm, D), lambda r: (r, 0)),
                  pl.BlockSpec((bm, D), lambda r: (r, 0)),
                  pl.BlockSpec((D, bm), lambda r: (0, r)),
                  pl.BlockSpec((nb, 1, D), lambda r: (0, 0, 0))],
        out_specs=pl.BlockSpec((bm, D), lambda r: (r, 0)),
        out_shape=jax.ShapeDtypeStruct((L, D), jnp.float32),
    )(eta_arr, hcur, h0, gdense, gscatter, bnd)


_NW = 32
_CH = 32


_GNB = 4


def _gather_rows(table, idx):
    V, D = table.shape
    N = idx.shape[0]
    per_w = N // _NW
    n_chunks = per_w // _CH
    mesh = plsc.VectorSubcoreMesh(core_axis_name="c", subcore_axis_name="s")

    @functools.partial(
        pl.kernel, mesh=mesh,
        out_type=jax.ShapeDtypeStruct((N, D), jnp.float32),
        scratch_types=[pltpu.VMEM((per_w,), jnp.int32)]
        + [pltpu.VMEM((_CH, D), jnp.float32)] * _GNB
        + [pltpu.SemaphoreType.DMA] * (2 * _GNB),
    )
    def k(table_hbm, idx_hbm, out_hbm, idx_v, *bufsem):
        bufs = bufsem[:_GNB]
        gsems = bufsem[_GNB:2 * _GNB]
        wsems = bufsem[2 * _GNB:]
        wid = lax.axis_index("s") * 2 + lax.axis_index("c")
        base = pl.multiple_of(wid * per_w, per_w)
        pltpu.sync_copy(idx_hbm.at[pl.ds(base, per_w)], idx_v)
        gcp, wcp = {}, {}
        for ci in range(n_chunks):
            if ci >= _GNB:
                wcp[ci - _GNB].wait()
            gcp[ci] = pltpu.async_copy(
                table_hbm.at[idx_v.at[pl.ds(ci * _CH, _CH)]],
                bufs[ci % _GNB], gsems[ci % _GNB])
            if ci >= 1:
                gcp[ci - 1].wait()
                wcp[ci - 1] = pltpu.async_copy(
                    bufs[(ci - 1) % _GNB],
                    out_hbm.at[pl.ds(base + (ci - 1) * _CH, _CH)],
                    wsems[(ci - 1) % _GNB])
        gcp[n_chunks - 1].wait()
        wcp[n_chunks - 1] = pltpu.async_copy(
            bufs[(n_chunks - 1) % _GNB],
            out_hbm.at[pl.ds(base + (n_chunks - 1) * _CH, _CH)],
            wsems[(n_chunks - 1) % _GNB])
        for ci in range(max(0, n_chunks - _GNB), n_chunks):
            wcp[ci].wait()

    return k(table, idx)


_SLC = 16
_SCE = 256


def _scatter_add_rows(gt4t, idx, L):
    G, D, Le = gt4t.shape
    n_slices = D // _SLC
    n_stage = Le // _SCE
    n_chunks = G * n_stage
    mesh = plsc.VectorSubcoreMesh(core_axis_name="c", subcore_axis_name="s")

    @functools.partial(
        pl.kernel, mesh=mesh,
        out_type=jax.ShapeDtypeStruct((D, L), jnp.float32),
        compiler_params=pltpu.CompilerParams(needs_layout_passes=False),
        scratch_types=[pltpu.VMEM((G * Le,), jnp.int32),
                       pltpu.VMEM((_SLC, _SCE), jnp.float32),
                       pltpu.VMEM((_SLC, _SCE), jnp.float32),
                       pltpu.VMEM((_SLC * L,), jnp.float32)]
        + [pltpu.SemaphoreType.DMA] * 6,
    )
    def k(rows_hbm, idx_hbm, out_hbm, idx_v, st0, st1, acc, s0, s1, *wsems):
        tid = lax.axis_index("s") * 2 + lax.axis_index("c")
        pltpu.sync_copy(idx_hbm, idx_v)
        stages = (st0, st1)
        ssems = (s0, s1)

        def do_slice(sl):
            c0 = pl.multiple_of(sl * _SLC, _SLC)

            def zero_body(j, carry):
                j16 = pl.multiple_of(j * 16, 16)
                acc[pl.ds(j16, 16)] = jnp.zeros((16,), jnp.float32)
                return carry

            lax.fori_loop(0, (_SLC * L) // 16, zero_body, 0)

            def start(ci, b):
                g = ci // n_stage
                e0 = (ci % n_stage) * _SCE
                pltpu.async_copy(
                    rows_hbm.at[g, pl.ds(c0, _SLC), pl.ds(e0, _SCE)],
                    stages[b], ssems[b])

            def process(ci, b):
                g = ci // n_stage
                e0 = (ci % n_stage) * _SCE
                stage = stages[b]
                off = pl.multiple_of(g * Le + e0, _SCE)

                def sub_body(sc, carry2):
                    s16 = pl.multiple_of(sc * 16, 16)
                    t16 = idx_v[pl.ds(off + s16, 16)]
                    for c in range(_SLC):
                        vals = stage[c, pl.ds(s16, 16)]
                        plsc.addupdate_scatter(acc, [c * L + t16], vals)
                    return carry2

                lax.fori_loop(0, _SCE // 16, sub_body, 0)

            start(0, 0)
            start(1, 1)

            def ring_body(it, carry):
                base = it * 2
                for b in range(2):
                    ci = base + b
                    pltpu.make_async_copy(
                        rows_hbm.at[0, pl.ds(0, _SLC), pl.ds(0, _SCE)],
                        stages[b], ssems[b]).wait()
                    process(ci, b)

                    @pl.when(ci + 2 < n_chunks)
                    def _():
                        start(ci + 2, b)
                return carry

            lax.fori_loop(0, n_chunks // 2, ring_body, 0)

            wcps = []
            for c in range(_SLC):
                if c >= 4:
                    wcps[c - 4].wait()
                wcps.append(pltpu.async_copy(
                    acc.at[pl.ds(c * L, L)], out_hbm.at[c0 + c],
                    wsems[c % 4]))
            for c in range(_SLC - 4, _SLC):
                wcps[c].wait()

        do_slice(tid)

        @pl.when(tid < n_slices - _NW)
        def _():
            do_slice(tid + _NW)

    return k(gt4t, idx)


def kernel(h, attention_weights, W1, b1, g1, beta1, W2, b2, g2, beta2, W3,
           b3, eta, pos_table):
    B, L, D = h.shape
    hid = W1.shape[1]
    maxrel = (pos_table.shape[0] - 1) // 2
    h2 = h.reshape(L, D)
    attn = attention_weights.reshape(L, L)

    vals8, idx8 = _topk(attn)
    w8, iij8, iji8 = _wix(vals8, idx8, maxrel)

    t4 = idx8.T[:4].reshape(-1)
    w8b = jnp.broadcast_to(w8.T[:5][:, :, None], (5, L, 8))

    bf = lambda a: a.astype(jnp.bfloat16)
    zb = jnp.zeros((1, hid), jnp.float32)
    A = _mm(h2, bf(W1[:D]), zb)
    Bv = _mm(h2, bf(W1[D:2 * D]), zb)
    P = _mm(pos_table, bf(W1[2 * D:]), b1.reshape(1, hid))
    p_ij = P[maxrel - 1:maxrel]
    p_ji = P[maxrel + 1:maxrel + 2]

    pij_tab = P[:maxrel]
    pji_tab = P[maxrel + 1:]
    split = lambda t: (bf(t), bf(t - bf(t).astype(jnp.float32)))
    pij_hi, pij_lo = split(pij_tab)
    pji_hi, pji_lo = split(pji_tab)

    Gh = _gather_rows(h2, t4)
    Gbv = _mm(Gh, bf(W1[D:2 * D]), zb)
    Ga = _mm(Gh, bf(W1[:D]), zb)

    vij = _mlp(A, Bv, Gbv, iij8, pij_hi, pij_lo, p_ij, bf(W2), b2, bf(W3),
               b3, g1, beta1, g2, beta2)
    vji = _mlp(Bv, A, Ga, iji8, pji_hi, pji_lo, p_ji, bf(W2), b2, bf(W3),
               b3, g1, beta1, g2, beta2)

    n_edges = 4 * L + (L - 1)
    inv_denom = 1.0 / (n_edges * D + 1e-8)
    eta_arr = eta.reshape(1)

    hcur = h2
    for step in range(3):
        HT4 = (Gh if step == 0 else _gather_rows(hcur, t4)).reshape(4, L, D)
        gdense, GT4T, bnd = _step_dense(hcur, HT4, vij, vji, w8b)
        gsc_t = _scatter_add_rows(GT4T, t4, L)
        hcur = _update(hcur, h2, gdense, gsc_t, bnd, eta_arr, inv_denom)
    return hcur.reshape(B, L, D)


# speedup vs baseline: 6.3939x; 1.0780x over previous
import functools

import jax
import jax.numpy as jnp
from jax import lax
from jax.experimental import pallas as pl
from jax.experimental.pallas import tpu as pltpu
from jax.experimental.pallas import tpu_sc as plsc

LAM = 0.01
NEG_INF = float("-inf")


def _topk_body(attn_ref, vals_ref, idx_ref, *, blk, L):
    r = pl.program_id(0)
    x = attn_ref[...]
    rows = r * blk + lax.broadcasted_iota(jnp.int32, (blk, L), 0)
    cols = lax.broadcasted_iota(jnp.int32, (blk, L), 1)
    cur = jnp.where(cols < rows, x, NEG_INF)
    vs, ids = [], []
    for _ in range(4):
        m = jnp.max(cur, axis=1, keepdims=True)
        cand = jnp.where(cur == m, cols, L)
        a = jnp.min(cand, axis=1, keepdims=True)
        a = jnp.where(a == L, 0, a)
        vs.append(m)
        ids.append(a)
        cur = jnp.where(cols == a, NEG_INF, cur)
    chain_v = jnp.sum(jnp.where(cols == rows - 1, x, 0.0), axis=1, keepdims=True)
    chain_t = jnp.maximum(rows[:, 0:1] - 1, 0)
    zf = jnp.zeros((blk, 1), jnp.float32)
    zi = jnp.zeros((blk, 1), jnp.int32)
    vals_ref[...] = jnp.concatenate(vs + [chain_v, zf, zf, zf], axis=1)
    idx_ref[...] = jnp.concatenate(ids + [chain_t, zi, zi, zi], axis=1)


def _topk(attn):
    L = attn.shape[0]
    blk = min(256, L)
    return pl.pallas_call(
        functools.partial(_topk_body, blk=blk, L=L),
        grid=(L // blk,),
        in_specs=[pl.BlockSpec((blk, L), lambda r: (r, 0))],
        out_specs=[pl.BlockSpec((blk, 8), lambda r: (r, 0)),
                   pl.BlockSpec((blk, 8), lambda r: (r, 0))],
        out_shape=[jax.ShapeDtypeStruct((L, 8), jnp.float32),
                   jax.ShapeDtypeStruct((L, 8), jnp.int32)],
    )(attn)


def _wix_body(vals_ref, idx_ref, w_ref, iij_ref, iji_ref, *, L, maxrel):
    vals = vals_ref[...]
    t8 = idx_ref[...]
    lane = lax.broadcasted_iota(jnp.int32, (L, 8), 1)
    rowv = lax.broadcasted_iota(jnp.int32, (L, 8), 0)
    m = jnp.max(vals, axis=0, keepdims=True)
    e = jnp.exp(vals - m)
    s = jnp.sum(e, axis=0, keepdims=True)
    wsm = e / s
    w_ref[...] = jnp.where(lane < 4, wsm, jnp.where(lane == 4, vals, 0.0))
    rel = t8 - rowv
    iij_ref[...] = jnp.clip(rel, -maxrel, maxrel) + maxrel
    iji_ref[...] = jnp.clip(-rel, -maxrel, maxrel) - 1


def _wix(vals8, idx8, maxrel):
    L = vals8.shape[0]
    return pl.pallas_call(
        functools.partial(_wix_body, L=L, maxrel=maxrel),
        in_specs=[pl.BlockSpec((L, 8), lambda: (0, 0))] * 2,
        out_specs=[pl.BlockSpec((L, 8), lambda: (0, 0))] * 3,
        out_shape=[jax.ShapeDtypeStruct((L, 8), jnp.float32),
                   jax.ShapeDtypeStruct((L, 8), jnp.int32),
                   jax.ShapeDtypeStruct((L, 8), jnp.int32)],
    )(vals8, idx8)


def _mm_body(x_ref, w_ref, b_ref, o_ref):
    x = x_ref[...].astype(jnp.bfloat16)
    o_ref[...] = jnp.dot(x, w_ref[...], preferred_element_type=jnp.float32) + b_ref[...]


def _mm(x, wbf, bias):
    M, K = x.shape
    N = wbf.shape[1]
    bm = min(256, M)
    return pl.pallas_call(
        _mm_body,
        grid=(pl.cdiv(M, bm),),
        in_specs=[pl.BlockSpec((bm, K), lambda r: (r, 0)),
                  pl.BlockSpec((K, N), lambda r: (0, 0)),
                  pl.BlockSpec((1, N), lambda r: (0, 0))],
        out_specs=pl.BlockSpec((bm, N), lambda r: (r, 0)),
        out_shape=jax.ShapeDtypeStruct((M, N), jnp.float32),
    )(x, wbf, bias)


def _ln(x, g, b):
    mu = jnp.mean(x, axis=-1, keepdims=True)
    v = jnp.mean((x - mu) ** 2, axis=-1, keepdims=True)
    return (x - mu) * lax.rsqrt(v + 1e-5) * g + b


def _mlp_body(base_ref, sh_ref, shp_ref, gt_ref, idx_ref, phi_ref, plo_ref,
              pch_ref, w2_ref, b2_ref, w3_ref, b3_ref, g1_ref, be1_ref,
              g2_ref, be2_ref, v_ref, *, bm, maxrel):
    g = pl.program_id(0)
    base = base_ref[...]
    lane8 = lax.broadcasted_iota(jnp.int32, (bm, 8), 1)
    sel = jnp.sum(jnp.where(lane8 == jnp.minimum(g, 3), idx_ref[...], 0),
                  axis=1)
    cols = lax.broadcasted_iota(jnp.int32, (bm, maxrel), 1)
    oh = (cols == sel[:, None]).astype(jnp.bfloat16)
    pe = (jnp.dot(oh, phi_ref[...], preferred_element_type=jnp.float32)
          + jnp.dot(oh, plo_ref[...], preferred_element_type=jnp.float32))
    x1_topk = base + gt_ref[0] + pe
    sh_shift = jnp.concatenate([shp_ref[bm - 1:bm, :], sh_ref[:bm - 1, :]],
                               axis=0)
    x1_chain = base + sh_shift + pch_ref[...]
    x1 = jnp.where(g == 4, x1_chain, x1_topk)
    u = jax.nn.relu(_ln(x1, g1_ref[...], be1_ref[...])).astype(jnp.bfloat16)
    h2 = jnp.dot(u, w2_ref[...], preferred_element_type=jnp.float32) + b2_ref[...]
    u2 = jax.nn.relu(_ln(h2, g2_ref[...], be2_ref[...])).astype(jnp.bfloat16)
    v = jnp.dot(u2, w3_ref[...], preferred_element_type=jnp.float32) + b3_ref[...]
    n = jnp.sqrt(jnp.sum(v * v, axis=-1, keepdims=True))
    v_ref[0] = v / jnp.maximum(n, 1e-8)


def _mlp(base, sh, gtab4, idx8, phi, plo, p_chain, w2bf, b2, w3bf, b3, g1,
         be1, g2, be2):
    L, hid = base.shape
    D = w3bf.shape[1]
    maxrel = phi.shape[0]
    bm = min(256, L)
    grid = (5, L // bm)
    vec = lambda a: a.reshape(1, -1)
    g3 = lambda g: jnp.minimum(g, 3)
    return pl.pallas_call(
        functools.partial(_mlp_body, bm=bm, maxrel=maxrel),
        grid=grid,
        in_specs=[pl.BlockSpec((bm, hid), lambda g, r: (r, 0)),
                  pl.BlockSpec((bm, hid), lambda g, r: (r, 0)),
                  pl.BlockSpec((bm, hid), lambda g, r: (jnp.maximum(r - 1, 0), 0)),
                  pl.BlockSpec((1, bm, hid), lambda g, r: (g3(g), r, 0)),
                  pl.BlockSpec((bm, 8), lambda g, r: (r, 0)),
                  pl.BlockSpec((maxrel, hid), lambda g, r: (0, 0)),
                  pl.BlockSpec((maxrel, hid), lambda g, r: (0, 0)),
                  pl.BlockSpec((1, hid), lambda g, r: (0, 0)),
                  pl.BlockSpec((hid, hid), lambda g, r: (0, 0)),
                  pl.BlockSpec((1, hid), lambda g, r: (0, 0)),
                  pl.BlockSpec((hid, D), lambda g, r: (0, 0)),
                  pl.BlockSpec((1, D), lambda g, r: (0, 0)),
                  pl.BlockSpec((1, hid), lambda g, r: (0, 0)),
                  pl.BlockSpec((1, hid), lambda g, r: (0, 0)),
                  pl.BlockSpec((1, hid), lambda g, r: (0, 0)),
                  pl.BlockSpec((1, hid), lambda g, r: (0, 0))],
        out_specs=pl.BlockSpec((1, bm, D), lambda g, r: (g, r, 0)),
        out_shape=jax.ShapeDtypeStruct((5, L, D), jnp.float32),
    )(base, sh, sh, gtab4.reshape(4, L, hid), idx8, phi, plo,
      p_chain, w2bf, vec(b2), w3bf, vec(b3), vec(g1), vec(be1), vec(g2),
      vec(be2))


def _step_body(h_ref, hp_ref, ht_ref, vij_ref, vji_ref, w_ref, gd_ref,
               gt_ref, bnd_ref, *, bm):
    g = pl.program_id(1)
    hs = h_ref[...]
    ht_chain = jnp.concatenate([hp_ref[bm - 1:bm, :], hs[:bm - 1, :]], axis=0)
    ht = jnp.where(g == 4, ht_chain, ht_ref[0])
    v1 = vij_ref[0]
    v2 = vji_ref[0]
    w = w_ref[0, :, 0:1]
    a = jnp.sum(v1 * hs, axis=-1, keepdims=True)
    b = jnp.sum(v2 * ht, axis=-1, keepdims=True)
    delta = w * (hs - 2.0 * a * v1 - ht + 2.0 * b * v2)
    c = jnp.sum(v1 * delta, axis=-1, keepdims=True)
    d = jnp.sum(v2 * delta, axis=-1, keepdims=True)
    gs = delta - 2.0 * c * v1
    gt = -(delta - 2.0 * d * v2)

    @pl.when(g < 4)
    def _():
        gt_ref[0] = gt.T

    @pl.when(g == 4)
    def _():
        bnd_ref[...] = gt[0:1].reshape(1, 1, -1)

    shup = jnp.concatenate([gt[1:], jnp.zeros_like(gt[0:1])], axis=0)
    contrib = gs + jnp.where(g == 4, shup, 0.0)

    @pl.when(g == 0)
    def _():
        gd_ref[...] = contrib

    @pl.when(g != 0)
    def _():
        gd_ref[...] += contrib


def _step_dense(hcur, HT4, vij, vji, w8b):
    L, D = hcur.shape
    bm = min(256, L)
    nb = L // bm
    g3 = lambda g: jnp.minimum(g, 3)
    return pl.pallas_call(
        functools.partial(_step_body, bm=bm),
        grid=(nb, 5),
        in_specs=[pl.BlockSpec((bm, D), lambda r, g: (r, 0)),
                  pl.BlockSpec((bm, D), lambda r, g: (jnp.maximum(r - 1, 0), 0)),
                  pl.BlockSpec((1, bm, D), lambda r, g: (g3(g), r, 0)),
                  pl.BlockSpec((1, bm, D), lambda r, g: (g, r, 0)),
                  pl.BlockSpec((1, bm, D), lambda r, g: (g, r, 0)),
                  pl.BlockSpec((1, bm, 8), lambda r, g: (g, r, 0))],
        out_specs=[pl.BlockSpec((bm, D), lambda r, g: (r, 0)),
                   pl.BlockSpec((1, D, bm), lambda r, g: (g3(g), 0, r)),
                   pl.BlockSpec((1, 1, D), lambda r, g: (r, 0, 0))],
        out_shape=[jax.ShapeDtypeStruct((L, D), jnp.float32),
                   jax.ShapeDtypeStruct((4, D, L), jnp.float32),
                   jax.ShapeDtypeStruct((nb, 1, D), jnp.float32)],
    )(hcur, hcur, HT4, vij, vji, w8b)


def _upd_body(eta_ref, h_ref, h0_ref, gd_ref, gsc_ref, bnd_ref, o_ref, *,
              inv_denom, bm, nb):
    r = pl.program_id(0)
    eta = eta_ref[0]
    bnd_next = bnd_ref[jnp.minimum(r + 1, nb - 1)]
    rowpos = lax.broadcasted_iota(jnp.int32, (bm, 1), 0)
    add = jnp.where((rowpos == bm - 1) & (r < nb - 1), bnd_next, 0.0)
    h = h_ref[...]
    g = (gd_ref[...] + gsc_ref[...].T + add) * inv_denom
    o_ref[...] = h - eta * (g + LAM * (h - h0_ref[...]))


def _update(hcur, h0, gdense, gscatter, bnd, eta_arr, inv_denom):
    L, D = hcur.shape
    bm = min(256, L)
    nb = L // bm
    return pl.pallas_call(
        functools.partial(_upd_body, inv_denom=inv_denom, bm=bm, nb=nb),
        grid=(nb,),
        in_specs=[pl.BlockSpec(memory_space=pltpu.SMEM),
                  pl.BlockSpec((bm, D), lambda r: (r, 0)),
                  pl.BlockSpec((bm, D), lambda r: (r, 0)),
                  pl.BlockSpec((bm, D), lambda r: (r, 0)),
                  pl.BlockSpec((D, bm), lambda r: (0, r)),
                  pl.BlockSpec((nb, 1, D), lambda r: (0, 0, 0))],
        out_specs=pl.BlockSpec((bm, D), lambda r: (r, 0)),
        out_shape=jax.ShapeDtypeStruct((L, D), jnp.float32),
    )(eta_arr, hcur, h0, gdense, gscatter, bnd)


_NW = 32
_CH = 32


_GNB = 4


def _gather_rows(table, idx):
    V, D = table.shape
    N = idx.shape[0]
    per_w = N // _NW
    n_chunks = per_w // _CH
    mesh = plsc.VectorSubcoreMesh(core_axis_name="c", subcore_axis_name="s")

    @functools.partial(
        pl.kernel, mesh=mesh,
        out_type=jax.ShapeDtypeStruct((N, D), jnp.float32),
        scratch_types=[pltpu.VMEM((per_w,), jnp.int32)]
        + [pltpu.VMEM((_CH, D), jnp.float32)] * _GNB
        + [pltpu.SemaphoreType.DMA] * (2 * _GNB),
    )
    def k(table_hbm, idx_hbm, out_hbm, idx_v, *bufsem):
        bufs = bufsem[:_GNB]
        gsems = bufsem[_GNB:2 * _GNB]
        wsems = bufsem[2 * _GNB:]
        wid = lax.axis_index("s") * 2 + lax.axis_index("c")
        base = pl.multiple_of(wid * per_w, per_w)
        pltpu.sync_copy(idx_hbm.at[pl.ds(base, per_w)], idx_v)
        gcp, wcp = {}, {}
        for ci in range(n_chunks):
            if ci >= _GNB:
                wcp[ci - _GNB].wait()
            gcp[ci] = pltpu.async_copy(
                table_hbm.at[idx_v.at[pl.ds(ci * _CH, _CH)]],
                bufs[ci % _GNB], gsems[ci % _GNB])
            if ci >= 1:
                gcp[ci - 1].wait()
                wcp[ci - 1] = pltpu.async_copy(
                    bufs[(ci - 1) % _GNB],
                    out_hbm.at[pl.ds(base + (ci - 1) * _CH, _CH)],
                    wsems[(ci - 1) % _GNB])
        gcp[n_chunks - 1].wait()
        wcp[n_chunks - 1] = pltpu.async_copy(
            bufs[(n_chunks - 1) % _GNB],
            out_hbm.at[pl.ds(base + (n_chunks - 1) * _CH, _CH)],
            wsems[(n_chunks - 1) % _GNB])
        for ci in range(max(0, n_chunks - _GNB), n_chunks):
            wcp[ci].wait()

    return k(table, idx)


_SLC = 24
_SCE = 256


def _scatter_add_rows(gt4t, idx, L):
    G, D, Le = gt4t.shape
    n_slices = D // _SLC
    n_stage = Le // _SCE
    n_chunks = G * n_stage
    mesh = plsc.VectorSubcoreMesh(core_axis_name="c", subcore_axis_name="s")

    @functools.partial(
        pl.kernel, mesh=mesh,
        out_type=jax.ShapeDtypeStruct((D, L), jnp.float32),
        compiler_params=pltpu.CompilerParams(needs_layout_passes=False),
        scratch_types=[pltpu.VMEM((G * Le,), jnp.int32),
                       pltpu.VMEM((_SLC, _SCE), jnp.float32),
                       pltpu.VMEM((_SLC, _SCE), jnp.float32),
                       pltpu.VMEM((_SLC * L,), jnp.float32)]
        + [pltpu.SemaphoreType.DMA] * 6,
    )
    def k(rows_hbm, idx_hbm, out_hbm, idx_v, st0, st1, acc, s0, s1, *wsems):
        tid = lax.axis_index("s") * 2 + lax.axis_index("c")
        pltpu.sync_copy(idx_hbm, idx_v)
        stages = (st0, st1)
        ssems = (s0, s1)

        def do_slice(sl):
            c0 = pl.multiple_of(sl * _SLC, _SLC)

            def zero_body(j, carry):
                j16 = pl.multiple_of(j * 16, 16)
                acc[pl.ds(j16, 16)] = jnp.zeros((16,), jnp.float32)
                return carry

            lax.fori_loop(0, (_SLC * L) // 16, zero_body, 0)

            def start(ci, b):
                g = ci // n_stage
                e0 = (ci % n_stage) * _SCE
                pltpu.async_copy(
                    rows_hbm.at[g, pl.ds(c0, _SLC), pl.ds(e0, _SCE)],
                    stages[b], ssems[b])

            def process(ci, b):
                g = ci // n_stage
                e0 = (ci % n_stage) * _SCE
                stage = stages[b]
                off = pl.multiple_of(g * Le + e0, _SCE)

                def sub_body(sc, carry2):
                    s16 = pl.multiple_of(sc * 16, 16)
                    t16 = idx_v[pl.ds(off + s16, 16)]
                    for c in range(_SLC):
                        vals = stage[c, pl.ds(s16, 16)]
                        plsc.addupdate_scatter(acc, [c * L + t16], vals)
                    return carry2

                lax.fori_loop(0, _SCE // 16, sub_body, 0)

            start(0, 0)
            start(1, 1)

            def ring_body(it, carry):
                base = it * 2
                for b in range(2):
                    ci = base + b
                    pltpu.make_async_copy(
                        rows_hbm.at[0, pl.ds(0, _SLC), pl.ds(0, _SCE)],
                        stages[b], ssems[b]).wait()
                    process(ci, b)

                    @pl.when(ci + 2 < n_chunks)
                    def _():
                        start(ci + 2, b)
                return carry

            lax.fori_loop(0, n_chunks // 2, ring_body, 0)

            wcps = []
            for c in range(_SLC):
                if c >= 4:
                    wcps[c - 4].wait()
                wcps.append(pltpu.async_copy(
                    acc.at[pl.ds(c * L, L)], out_hbm.at[c0 + c],
                    wsems[c % 4]))
            for c in range(_SLC - 4, _SLC):
                wcps[c].wait()

        do_slice(tid)

        @pl.when(tid < n_slices - _NW)
        def _():
            do_slice(tid + _NW)

    return k(gt4t, idx)


def kernel(h, attention_weights, W1, b1, g1, beta1, W2, b2, g2, beta2, W3,
           b3, eta, pos_table):
    B, L, D = h.shape
    hid = W1.shape[1]
    maxrel = (pos_table.shape[0] - 1) // 2
    h2 = h.reshape(L, D)
    attn = attention_weights.reshape(L, L)

    vals8, idx8 = _topk(attn)
    w8, iij8, iji8 = _wix(vals8, idx8, maxrel)

    t4 = idx8.T[:4].reshape(-1)
    w8b = jnp.broadcast_to(w8.T[:5][:, :, None], (5, L, 8))

    bf = lambda a: a.astype(jnp.bfloat16)
    zb = jnp.zeros((1, hid), jnp.float32)
    A = _mm(h2, bf(W1[:D]), zb)
    Bv = _mm(h2, bf(W1[D:2 * D]), zb)
    P = _mm(pos_table, bf(W1[2 * D:]), b1.reshape(1, hid))
    p_ij = P[maxrel - 1:maxrel]
    p_ji = P[maxrel + 1:maxrel + 2]

    pij_tab = P[:maxrel]
    pji_tab = P[maxrel + 1:]
    split = lambda t: (bf(t), bf(t - bf(t).astype(jnp.float32)))
    pij_hi, pij_lo = split(pij_tab)
    pji_hi, pji_lo = split(pji_tab)

    Gh = _gather_rows(h2, t4)
    Gbv = _mm(Gh, bf(W1[D:2 * D]), zb)
    Ga = _mm(Gh, bf(W1[:D]), zb)

    vij = _mlp(A, Bv, Gbv, iij8, pij_hi, pij_lo, p_ij, bf(W2), b2, bf(W3),
               b3, g1, beta1, g2, beta2)
    vji = _mlp(Bv, A, Ga, iji8, pji_hi, pji_lo, p_ji, bf(W2), b2, bf(W3),
               b3, g1, beta1, g2, beta2)

    n_edges = 4 * L + (L - 1)
    inv_denom = 1.0 / (n_edges * D + 1e-8)
    eta_arr = eta.reshape(1)

    hcur = h2
    for step in range(3):
        HT4 = (Gh if step == 0 else _gather_rows(hcur, t4)).reshape(4, L, D)
        gdense, GT4T, bnd = _step_dense(hcur, HT4, vij, vji, w8b)
        gsc_t = _scatter_add_rows(GT4T, t4, L)
        hcur = _update(hcur, h2, gdense, gsc_t, bnd, eta_arr, inv_denom)
    return hcur.reshape(B, L, D)
